# Initial kernel scaffold; baseline (speedup 1.0000x reference)
import functools

import jax
import jax.numpy as jnp
from jax import lax
from jax.experimental import pallas as pl
from jax.experimental.pallas import tpu as pltpu
from jax.experimental.pallas import tpu_sc as plsc

N = 4096
E = 131072
THR = 0.1
BM = 512
NB = N // BM
F32 = jnp.float32


NTILES = 16
EPT = E // NTILES
NCHUNKS = 16
CROWS = N // NCHUNKS
CWORDS = CROWS * N
DUMPW = 2048
ZN = CWORDS // NTILES
IDX_ROWS = EPT // 128


def _count_body(edges_hbm, out_hbm, row_v, col_v, idx_v, ones_v, zer_v, buf_sh):
    cid = lax.axis_index("c")
    sid = lax.axis_index("s")

    ebase = pl.multiple_of(sid * EPT, 8)
    pltpu.sync_copy(edges_hbm.at[0, pl.ds(ebase, EPT)], row_v)
    pltpu.sync_copy(edges_hbm.at[1, pl.ds(ebase, EPT)], col_v)

    for j in range(8):
        ones_v[pl.ds(j * 16, 16)] = jnp.ones((16,), F32)

    def zbody(i, _):
        zer_v[pl.ds(i * 16, 16)] = jnp.zeros((16,), F32)
        return 0

    lax.fori_loop(0, ZN // 16, zbody, 0)

    lane = lax.iota(jnp.int32, 16)

    for ch in range(NCHUNKS // 2):
        chunk = cid * (NCHUNKS // 2) + ch
        base = pl.multiple_of(chunk * CWORDS, 8)

        pltpu.sync_copy(zer_v, buf_sh.at[pl.ds(pl.multiple_of(sid * ZN, 8), ZN)])

        @pl.when(sid == 0)
        def _zd():
            pltpu.sync_copy(zer_v.at[pl.ds(0, DUMPW)],
                            buf_sh.at[pl.ds(CWORDS, DUMPW)])

        def ibody(i, _):
            for j in range(8):
                e0 = i * 128 + j * 16
                r = row_v[pl.ds(e0, 16)]
                c = col_v[pl.ds(e0, 16)]
                full = r * N + c - base
                valid = (full >= 0) & (full < CWORDS)
                dump = CWORDS + ((e0 + lane) & (DUMPW - 1))
                idx_v[i, pl.ds(j * 16, 16)] = jnp.where(valid, full, dump)
            return 0

        lax.fori_loop(0, IDX_ROWS, ibody, 0)

        plsc.subcore_barrier()

        for jj in range(IDX_ROWS):
            pltpu.sync_copy(ones_v, buf_sh.at[idx_v.at[jj]], add=True)

        plsc.subcore_barrier()

        dst = pl.multiple_of(base + sid * ZN, 8)
        pltpu.sync_copy(buf_sh.at[pl.ds(pl.multiple_of(sid * ZN, 8), ZN)],
                        out_hbm.at[pl.ds(dst, ZN)])


def _build_count(edge_index):
    mesh = plsc.VectorSubcoreMesh(core_axis_name="c", subcore_axis_name="s")
    fn = pl.kernel(
        _count_body,
        out_type=jax.ShapeDtypeStruct((N * N,), F32),
        mesh=mesh,
        scratch_types=[
            pltpu.VMEM((EPT,), jnp.int32),
            pltpu.VMEM((EPT,), jnp.int32),
            pltpu.VMEM((IDX_ROWS, 128), jnp.int32),
            pltpu.VMEM((128,), F32),
            pltpu.VMEM((ZN,), F32),
            pltpu.VMEM_SHARED((CWORDS + DUMPW,), F32),
        ],
    )
    return fn(edge_index)


def _rownorm_body(x_ref, o_ref):
    x = x_ref[...]
    nrm = jnp.maximum(jnp.sqrt(jnp.sum(x * x, axis=1, keepdims=True)), 1e-8)
    o_ref[...] = x / nrm


def _rownorm(x):
    n, d = x.shape
    return pl.pallas_call(
        _rownorm_body,
        grid=(n // BM,),
        in_specs=[pl.BlockSpec((BM, d), lambda i: (i, 0))],
        out_specs=pl.BlockSpec((BM, d), lambda i: (i, 0)),
        out_shape=jax.ShapeDtypeStruct((n, d), F32),
    )(x)


def _simadj_body(*refs, with_prev, with_h):
    idx = 0
    xi = refs[idx]; idx += 1
    xj = refs[idx]; idx += 1
    if with_prev:
        pi = refs[idx]; idx += 1
        pj = refs[idx]; idx += 1
    cref = refs[idx]; idx += 1
    if with_h:
        xr = refs[idx]; idx += 1
        wr = refs[idx]; idx += 1
    a_ref = refs[idx]; idx += 1
    dinv_ref = refs[idx]; idx += 1
    if with_h:
        hpack_ref = refs[idx]; idx += 1
    acc_ref = refs[idx]

    i = pl.program_id(0)
    j = pl.program_id(1)
    s = jnp.dot(xi[...], xj[...].T, preferred_element_type=F32)
    ri = lax.broadcasted_iota(jnp.int32, (BM, BM), 0) + i * BM
    ci = lax.broadcasted_iota(jnp.int32, (BM, BM), 1) + j * BM
    m = (s >= THR) & (ri != ci)
    if with_prev:
        sp = jnp.dot(pi[...], pj[...].T, preferred_element_type=F32)
        m = m & (sp >= THR)
    a = jnp.where(m, s, 0.0) * cref[...]
    a_ref[...] = a

    @pl.when(j == 0)
    def _z():
        acc_ref[...] = jnp.zeros_like(acc_ref)

    acc_ref[...] += jnp.sum(a, axis=1, keepdims=True)

    @pl.when(j == NB - 1)
    def _f():
        dinv_ref[...] = lax.rsqrt(acc_ref[...] + 1.0)
        if with_h:
            hpack_ref[...] = jnp.dot(xr[...], wr[...],
                                     preferred_element_type=F32)


def _simadj(xn, c, prev=None, hx=None, wcat=None):
    with_prev = prev is not None
    with_h = hx is not None
    args = [xn, xn]
    in_specs = [pl.BlockSpec((BM, xn.shape[1]), lambda i, j: (i, 0)),
                pl.BlockSpec((BM, xn.shape[1]), lambda i, j: (j, 0))]
    if with_prev:
        args += [prev, prev]
        in_specs += [pl.BlockSpec((BM, prev.shape[1]), lambda i, j: (i, 0)),
                     pl.BlockSpec((BM, prev.shape[1]), lambda i, j: (j, 0))]
    args.append(c)
    in_specs.append(pl.BlockSpec((BM, BM), lambda i, j: (i, j)))
    out_shapes = [jax.ShapeDtypeStruct((N, N), F32),
                  jax.ShapeDtypeStruct((N, 1), F32)]
    out_specs = [pl.BlockSpec((BM, BM), lambda i, j: (i, j)),
                 pl.BlockSpec((BM, 1), lambda i, j: (i, 0))]
    if with_h:
        args += [hx, wcat]
        in_specs += [pl.BlockSpec((BM, hx.shape[1]), lambda i, j: (i, 0)),
                     pl.BlockSpec(wcat.shape, lambda i, j: (0, 0))]
        out_shapes.append(jax.ShapeDtypeStruct((N, wcat.shape[1]), F32))
        out_specs.append(pl.BlockSpec((BM, wcat.shape[1]), lambda i, j: (i, 0)))
    return pl.pallas_call(
        functools.partial(_simadj_body, with_prev=with_prev, with_h=with_h),
        grid=(NB, NB),
        in_specs=in_specs,
        out_specs=out_specs,
        out_shape=out_shapes,
        scratch_shapes=[pltpu.VMEM((BM, 1), F32)],
    )(*args)


def _adjmm_body(*refs, n_extra, n_out, epilogue, bscale):
    idx = 0
    p_ref = refs[idx]; idx += 1
    b_ref = refs[idx]; idx += 1
    if bscale:
        bs_ref = refs[idx]; idx += 1
    extra_refs = refs[idx:idx + n_extra]; idx += n_extra
    out_refs = refs[idx:idx + n_out]; idx += n_out
    acc_ref = refs[idx]

    k = pl.program_id(1)

    @pl.when(k == 0)
    def _z():
        acc_ref[...] = jnp.zeros_like(acc_ref)

    bb = b_ref[...]
    if bscale:
        bb = bs_ref[...] * bb
    acc_ref[...] += jnp.dot(p_ref[...], bb, preferred_element_type=F32)

    @pl.when(k == NB - 1)
    def _f():
        outs = epilogue(acc_ref[...], [er[...] for er in extra_refs])
        for o_ref, o in zip(out_refs, outs):
            o_ref[...] = o


def _adjmm(p, b, extras, epilogue, out_widths, bscale=None):
    w = b.shape[1]
    args = [p, b]
    in_specs = [pl.BlockSpec((BM, BM), lambda i, k: (i, k)),
                pl.BlockSpec((BM, w), lambda i, k: (k, 0))]
    if bscale is not None:
        args.append(bscale)
        in_specs.append(pl.BlockSpec((BM, 1), lambda i, k: (k, 0)))
    for arr, mode in extras:
        args.append(arr)
        if mode == "i":
            in_specs.append(pl.BlockSpec((BM, arr.shape[1]),
                                         lambda i, k: (i, 0)))
        else:
            in_specs.append(pl.BlockSpec(arr.shape, lambda i, k: (0, 0)))
    out_shapes = [jax.ShapeDtypeStruct((N, ow), F32) for ow in out_widths]
    out_specs = [pl.BlockSpec((BM, ow), lambda i, k: (i, 0))
                 for ow in out_widths]
    res = pl.pallas_call(
        functools.partial(_adjmm_body, n_extra=len(extras),
                          n_out=len(out_widths), epilogue=epilogue,
                          bscale=bscale is not None),
        grid=(NB, NB),
        in_specs=in_specs,
        out_specs=out_specs,
        out_shape=out_shapes,
        scratch_shapes=[pltpu.VMEM((BM, w), F32)],
    )(*args)
    return res


def kernel(x, adj_l, dec, edge_index, Wse1, bse1, Wse2, bse2, Wsd1, bsd1,
           Wsd2, bsd2, We1, be1, We2, be2, Wg3, Wg4, Wg5, weight1, weight2,
           cluster_layer):
    C = _build_count(edge_index).reshape(N, N)
    xn = _rownorm(x)

    wcat = jnp.concatenate([Wse1, We1], axis=1)
    A1, dinv1, hpack = _simadj(xn, C, hx=x, wcat=wcat)
    h = hpack[:, :256]
    u1 = hpack[:, 256:]

    Wse2p = jnp.pad(Wse2, ((0, 0), (0, 96)))
    bse1r = bse1.reshape(1, -1)

    def ep1(acc, ex):
        dinv, hh, bb, w2p = ex
        h1 = jnp.maximum(dinv * acc + dinv * dinv * hh + bb, 0.0)
        nrm = jnp.maximum(jnp.sqrt(jnp.sum(h1 * h1, axis=1, keepdims=True)),
                          1e-8)
        return h1 / nrm, jnp.dot(h1, w2p, preferred_element_type=F32)

    h1n, h2p = _adjmm(A1, h,
                      [(dinv1, "i"), (h, "i"), (bse1r, "full"),
                       (Wse2p, "full")],
                      ep1, [256, 128], bscale=dinv1)

    A2, dinv2 = _simadj(h1n, C, prev=xn)
    bse2p = jnp.pad(bse2, (0, 96)).reshape(1, -1)
    Wsd1p = jnp.pad(Wsd1, ((0, 96), (0, 0)))

    def ep2(acc, ex):
        dinv, hh, bb, wsd1p = ex
        e = dinv * acc + dinv * dinv * hh + bb
        return e, jnp.dot(e, wsd1p, preferred_element_type=F32)

    embp, g1 = _adjmm(A2, h2p,
                      [(dinv2, "i"), (h2p, "i"), (bse2p, "full"),
                       (Wsd1p, "full")],
                      ep2, [128, 256], bscale=dinv2)

    B1 = jnp.concatenate([g1, u1], axis=1)
    bias3 = jnp.concatenate([bsd1, be1]).reshape(1, -1)
    W6 = jnp.zeros((384, 384), F32)
    W6 = W6.at[:256, :256].set(Wsd2)
    W6 = W6.at[256:, 256:288].set(We2)

    def ep3(acc, ex):
        bb, w6 = ex
        t = jnp.maximum(acc + bb, 0.0)
        return (jnp.dot(t, w6, preferred_element_type=F32),)

    (B2,) = _adjmm(C, B1, [(bias3, "full"), (W6, "full")], ep3, [384])

    bias4 = jnp.concatenate([bsd2, be2, jnp.zeros((96,), F32)]).reshape(1, -1)
    W7z = jnp.zeros((384, 128), F32).at[256:288, 0:64].set(Wg3)
    W7d = jnp.zeros((128, 128), F32).at[:, 64:72].set(weight1)

    def ep4(acc, ex):
        bb, w7z, w7d, dcb = ex
        z2 = acc + bb
        p1 = (jnp.dot(z2, w7z, preferred_element_type=F32)
              + jnp.dot(dcb, w7d, preferred_element_type=F32))
        return z2, p1

    Z2, P1 = _adjmm(C, B2,
                    [(bias4, "full"), (W7z, "full"), (W7d, "full"),
                     (dec, "i")],
                    ep4, [384, 128])

    W8 = jnp.zeros((128, 256), F32)
    W8 = W8.at[0:64, 0:128].set(Wg4)
    W8 = W8.at[64:72, 128:256].set(weight2)

    def ep5(acc, ex):
        (w8,) = ex
        colid = lax.broadcasted_iota(jnp.int32, acc.shape, 1)
        q1 = jnp.where(colid < 64, jnp.maximum(acc, 0.0), acc)
        return q1, jnp.dot(q1, w8, preferred_element_type=F32)

    Q1, P2 = _adjmm(adj_l, P1, [(W8, "full")], ep5, [128, 256])

    W9 = jnp.zeros((256, 256), F32).at[0:128, :].set(Wg5)

    def ep6(acc, ex):
        (w9,) = ex
        colid = lax.broadcasted_iota(jnp.int32, acc.shape, 1)
        q2 = jnp.where(colid < 128, jnp.maximum(acc, 0.0), acc)
        return q2, jnp.dot(q2, w9, preferred_element_type=F32)

    Q2, p3 = _adjmm(adj_l, P2, [(W9, "full")], ep6, [256, 256])

    emb_gcn = embp[:, :32]
    h_enc = Z2[:, 256:288]
    emb_d = Q1[:, 64:72]
    z = jnp.concatenate([h_enc, emb_gcn, emb_d], axis=1)
    z_pad = jnp.pad(z, ((0, 0), (0, 56)))
    cl_pad = jnp.pad(cluster_layer, ((0, 6), (0, 56)))

    def ep7(acc, ex):
        zp, cl = ex
        z2s = jnp.sum(zp * zp, axis=1, keepdims=True)
        c2s = jnp.sum(cl * cl, axis=1)[None, :]
        cross = jnp.dot(zp, cl.T, preferred_element_type=F32)
        dist = z2s - 2.0 * cross + c2s
        qv = 1.0 / (1.0 + dist + 1e-8)
        qv = qv * qv / 2.0
        colid = lax.broadcasted_iota(jnp.int32, qv.shape, 1)
        qv = jnp.where(colid < 10, qv, 0.0)
        qn = qv / jnp.sum(qv, axis=1, keepdims=True)
        return acc, qn

    z_hat, qfull = _adjmm(adj_l, p3, [(z_pad, "i"), (cl_pad, "full")],
                          ep7, [256, 16])

    return (z_hat, qfull[:, :10], z, Z2[:, :256], Q2[:, 128:])


# speedup vs baseline: 13.2990x; 13.2990x over previous
import functools

import jax
import jax.numpy as jnp
from jax import lax
from jax.experimental import pallas as pl
from jax.experimental.pallas import tpu as pltpu
from jax.experimental.pallas import tpu_sc as plsc

N = 4096
E = 131072
THR = 0.1
BM = 512
NB = N // BM
F32 = jnp.float32


NTILES = 16
EPT = E // NTILES
NCHUNKS = 16
CROWS = N // NCHUNKS
CWORDS = CROWS * N
DUMPW = 2048
ZN = CWORDS // NTILES
ZB = 8192
IDX_ROWS = EPT // 128


def _count_body(edges_hbm, out_hbm, row_v, col_v, idx_v, ones_v, zer_v, buf_sh):
    cid = lax.axis_index("c")
    sid = lax.axis_index("s")

    ebase = pl.multiple_of(sid * EPT, 8)
    pltpu.sync_copy(edges_hbm.at[0, pl.ds(ebase, EPT)], row_v)
    pltpu.sync_copy(edges_hbm.at[1, pl.ds(ebase, EPT)], col_v)

    for j in range(8):
        ones_v[pl.ds(j * 16, 16)] = jnp.ones((16,), F32)

    def zbody(i, _):
        zer_v[pl.ds(i * 16, 16)] = jnp.zeros((16,), F32)
        return 0

    lax.fori_loop(0, ZB // 16, zbody, 0)

    lane = lax.iota(jnp.int32, 16)

    for ch in range(NCHUNKS // 2):
        chunk = cid * (NCHUNKS // 2) + ch
        base = pl.multiple_of(chunk * CWORDS, 8)

        for zz in range(ZN // ZB):
            pltpu.sync_copy(
                zer_v,
                buf_sh.at[pl.ds(pl.multiple_of(sid * ZN + zz * ZB, 8), ZB)])

        @pl.when(sid == 0)
        def _zd():
            pltpu.sync_copy(zer_v.at[pl.ds(0, DUMPW)],
                            buf_sh.at[pl.ds(CWORDS, DUMPW)])

        def ibody(i, _):
            for j in range(8):
                e0 = i * 128 + j * 16
                r = row_v[pl.ds(e0, 16)]
                c = col_v[pl.ds(e0, 16)]
                full = r * N + c - base
                valid = (full >= 0) & (full < CWORDS)
                dump = CWORDS + ((e0 + lane) & (DUMPW - 1))
                idx_v[i, pl.ds(j * 16, 16)] = jnp.where(valid, full, dump)
            return 0

        lax.fori_loop(0, IDX_ROWS, ibody, 0)

        plsc.subcore_barrier()

        for jj in range(IDX_ROWS):
            pltpu.sync_copy(ones_v, buf_sh.at[idx_v.at[jj]], add=True)

        plsc.subcore_barrier()

        dst = pl.multiple_of(base + sid * ZN, 8)
        pltpu.sync_copy(buf_sh.at[pl.ds(pl.multiple_of(sid * ZN, 8), ZN)],
                        out_hbm.at[pl.ds(dst, ZN)])


def _build_count(edge_index):
    mesh = plsc.VectorSubcoreMesh(core_axis_name="c", subcore_axis_name="s")
    fn = pl.kernel(
        _count_body,
        out_type=jax.ShapeDtypeStruct((N * N,), F32),
        mesh=mesh,
        scratch_types=[
            pltpu.VMEM((EPT,), jnp.int32),
            pltpu.VMEM((EPT,), jnp.int32),
            pltpu.VMEM((IDX_ROWS, 128), jnp.int32),
            pltpu.VMEM((128,), F32),
            pltpu.VMEM((ZB,), F32),
            pltpu.VMEM_SHARED((CWORDS + DUMPW,), F32),
        ],
    )
    return fn(edge_index)


def _rownorm_body(x_ref, o_ref):
    x = x_ref[...]
    nrm = jnp.maximum(jnp.sqrt(jnp.sum(x * x, axis=1, keepdims=True)), 1e-8)
    o_ref[...] = x / nrm


def _rownorm(x):
    n, d = x.shape
    return pl.pallas_call(
        _rownorm_body,
        grid=(n // BM,),
        in_specs=[pl.BlockSpec((BM, d), lambda i: (i, 0))],
        out_specs=pl.BlockSpec((BM, d), lambda i: (i, 0)),
        out_shape=jax.ShapeDtypeStruct((n, d), F32),
    )(x)


def _simadj_body(*refs, with_prev, with_h):
    idx = 0
    xi = refs[idx]; idx += 1
    xj = refs[idx]; idx += 1
    if with_prev:
        pi = refs[idx]; idx += 1
        pj = refs[idx]; idx += 1
    cref = refs[idx]; idx += 1
    if with_h:
        xr = refs[idx]; idx += 1
        wr = refs[idx]; idx += 1
    a_ref = refs[idx]; idx += 1
    dinv_ref = refs[idx]; idx += 1
    if with_h:
        hpack_ref = refs[idx]; idx += 1
    acc_ref = refs[idx]

    i = pl.program_id(0)
    j = pl.program_id(1)
    s = jnp.dot(xi[...], xj[...].T, preferred_element_type=F32)
    ri = lax.broadcasted_iota(jnp.int32, (BM, BM), 0) + i * BM
    ci = lax.broadcasted_iota(jnp.int32, (BM, BM), 1) + j * BM
    m = (s >= THR) & (ri != ci)
    if with_prev:
        sp = jnp.dot(pi[...], pj[...].T, preferred_element_type=F32)
        m = m & (sp >= THR)
    a = jnp.where(m, s, 0.0) * cref[...]
    a_ref[...] = a

    @pl.when(j == 0)
    def _z():
        acc_ref[...] = jnp.zeros_like(acc_ref)

    acc_ref[...] += jnp.sum(a, axis=1, keepdims=True)

    @pl.when(j == NB - 1)
    def _f():
        dinv_ref[...] = lax.rsqrt(acc_ref[...] + 1.0)
        if with_h:
            hpack_ref[...] = jnp.dot(xr[...], wr[...],
                                     preferred_element_type=F32)


def _simadj(xn, c, prev=None, hx=None, wcat=None):
    with_prev = prev is not None
    with_h = hx is not None
    args = [xn, xn]
    in_specs = [pl.BlockSpec((BM, xn.shape[1]), lambda i, j: (i, 0)),
                pl.BlockSpec((BM, xn.shape[1]), lambda i, j: (j, 0))]
    if with_prev:
        args += [prev, prev]
        in_specs += [pl.BlockSpec((BM, prev.shape[1]), lambda i, j: (i, 0)),
                     pl.BlockSpec((BM, prev.shape[1]), lambda i, j: (j, 0))]
    args.append(c)
    in_specs.append(pl.BlockSpec((BM, BM), lambda i, j: (i, j)))
    out_shapes = [jax.ShapeDtypeStruct((N, N), F32),
                  jax.ShapeDtypeStruct((N, 1), F32)]
    out_specs = [pl.BlockSpec((BM, BM), lambda i, j: (i, j)),
                 pl.BlockSpec((BM, 1), lambda i, j: (i, 0))]
    if with_h:
        args += [hx, wcat]
        in_specs += [pl.BlockSpec((BM, hx.shape[1]), lambda i, j: (i, 0)),
                     pl.BlockSpec(wcat.shape, lambda i, j: (0, 0))]
        out_shapes.append(jax.ShapeDtypeStruct((N, wcat.shape[1]), F32))
        out_specs.append(pl.BlockSpec((BM, wcat.shape[1]), lambda i, j: (i, 0)))
    return pl.pallas_call(
        functools.partial(_simadj_body, with_prev=with_prev, with_h=with_h),
        grid=(NB, NB),
        in_specs=in_specs,
        out_specs=out_specs,
        out_shape=out_shapes,
        scratch_shapes=[pltpu.VMEM((BM, 1), F32)],
    )(*args)


def _adjmm_body(*refs, n_extra, n_out, epilogue, bscale):
    idx = 0
    p_ref = refs[idx]; idx += 1
    b_ref = refs[idx]; idx += 1
    if bscale:
        bs_ref = refs[idx]; idx += 1
    extra_refs = refs[idx:idx + n_extra]; idx += n_extra
    out_refs = refs[idx:idx + n_out]; idx += n_out
    acc_ref = refs[idx]

    k = pl.program_id(1)

    @pl.when(k == 0)
    def _z():
        acc_ref[...] = jnp.zeros_like(acc_ref)

    bb = b_ref[...]
    if bscale:
        bb = bs_ref[...] * bb
    acc_ref[...] += jnp.dot(p_ref[...], bb, preferred_element_type=F32)

    @pl.when(k == NB - 1)
    def _f():
        outs = epilogue(acc_ref[...], [er[...] for er in extra_refs])
        for o_ref, o in zip(out_refs, outs):
            o_ref[...] = o


def _adjmm(p, b, extras, epilogue, out_widths, bscale=None):
    w = b.shape[1]
    args = [p, b]
    in_specs = [pl.BlockSpec((BM, BM), lambda i, k: (i, k)),
                pl.BlockSpec((BM, w), lambda i, k: (k, 0))]
    if bscale is not None:
        args.append(bscale)
        in_specs.append(pl.BlockSpec((BM, 1), lambda i, k: (k, 0)))
    for arr, mode in extras:
        args.append(arr)
        if mode == "i":
            in_specs.append(pl.BlockSpec((BM, arr.shape[1]),
                                         lambda i, k: (i, 0)))
        else:
            in_specs.append(pl.BlockSpec(arr.shape, lambda i, k: (0, 0)))
    out_shapes = [jax.ShapeDtypeStruct((N, ow), F32) for ow in out_widths]
    out_specs = [pl.BlockSpec((BM, ow), lambda i, k: (i, 0))
                 for ow in out_widths]
    res = pl.pallas_call(
        functools.partial(_adjmm_body, n_extra=len(extras),
                          n_out=len(out_widths), epilogue=epilogue,
                          bscale=bscale is not None),
        grid=(NB, NB),
        in_specs=in_specs,
        out_specs=out_specs,
        out_shape=out_shapes,
        scratch_shapes=[pltpu.VMEM((BM, w), F32)],
    )(*args)
    return res


def kernel(x, adj_l, dec, edge_index, Wse1, bse1, Wse2, bse2, Wsd1, bsd1,
           Wsd2, bsd2, We1, be1, We2, be2, Wg3, Wg4, Wg5, weight1, weight2,
           cluster_layer):
    C = _build_count(edge_index).reshape(N, N)
    xn = _rownorm(x)

    wcat = jnp.concatenate([Wse1, We1], axis=1)
    A1, dinv1, hpack = _simadj(xn, C, hx=x, wcat=wcat)
    h = hpack[:, :256]
    u1 = hpack[:, 256:]

    Wse2p = jnp.pad(Wse2, ((0, 0), (0, 96)))
    bse1r = bse1.reshape(1, -1)

    def ep1(acc, ex):
        dinv, hh, bb, w2p = ex
        h1 = jnp.maximum(dinv * acc + dinv * dinv * hh + bb, 0.0)
        nrm = jnp.maximum(jnp.sqrt(jnp.sum(h1 * h1, axis=1, keepdims=True)),
                          1e-8)
        return h1 / nrm, jnp.dot(h1, w2p, preferred_element_type=F32)

    h1n, h2p = _adjmm(A1, h,
                      [(dinv1, "i"), (h, "i"), (bse1r, "full"),
                       (Wse2p, "full")],
                      ep1, [256, 128], bscale=dinv1)

    A2, dinv2 = _simadj(h1n, C, prev=xn)
    bse2p = jnp.pad(bse2, (0, 96)).reshape(1, -1)
    Wsd1p = jnp.pad(Wsd1, ((0, 96), (0, 0)))

    def ep2(acc, ex):
        dinv, hh, bb, wsd1p = ex
        e = dinv * acc + dinv * dinv * hh + bb
        return e, jnp.dot(e, wsd1p, preferred_element_type=F32)

    embp, g1 = _adjmm(A2, h2p,
                      [(dinv2, "i"), (h2p, "i"), (bse2p, "full"),
                       (Wsd1p, "full")],
                      ep2, [128, 256], bscale=dinv2)

    B1 = jnp.concatenate([g1, u1], axis=1)
    bias3 = jnp.concatenate([bsd1, be1]).reshape(1, -1)
    W6 = jnp.zeros((384, 384), F32)
    W6 = W6.at[:256, :256].set(Wsd2)
    W6 = W6.at[256:, 256:288].set(We2)

    def ep3(acc, ex):
        bb, w6 = ex
        t = jnp.maximum(acc + bb, 0.0)
        return (jnp.dot(t, w6, preferred_element_type=F32),)

    (B2,) = _adjmm(C, B1, [(bias3, "full"), (W6, "full")], ep3, [384])

    bias4 = jnp.concatenate([bsd2, be2, jnp.zeros((96,), F32)]).reshape(1, -1)
    W7z = jnp.zeros((384, 128), F32).at[256:288, 0:64].set(Wg3)
    W7d = jnp.zeros((128, 128), F32).at[:, 64:72].set(weight1)

    def ep4(acc, ex):
        bb, w7z, w7d, dcb = ex
        z2 = acc + bb
        p1 = (jnp.dot(z2, w7z, preferred_element_type=F32)
              + jnp.dot(dcb, w7d, preferred_element_type=F32))
        return z2, p1

    Z2, P1 = _adjmm(C, B2,
                    [(bias4, "full"), (W7z, "full"), (W7d, "full"),
                     (dec, "i")],
                    ep4, [384, 128])

    W8 = jnp.zeros((128, 256), F32)
    W8 = W8.at[0:64, 0:128].set(Wg4)
    W8 = W8.at[64:72, 128:256].set(weight2)

    def ep5(acc, ex):
        (w8,) = ex
        colid = lax.broadcasted_iota(jnp.int32, acc.shape, 1)
        q1 = jnp.where(colid < 64, jnp.maximum(acc, 0.0), acc)
        return q1, jnp.dot(q1, w8, preferred_element_type=F32)

    Q1, P2 = _adjmm(adj_l, P1, [(W8, "full")], ep5, [128, 256])

    W9 = jnp.zeros((256, 256), F32).at[0:128, :].set(Wg5)

    def ep6(acc, ex):
        (w9,) = ex
        colid = lax.broadcasted_iota(jnp.int32, acc.shape, 1)
        q2 = jnp.where(colid < 128, jnp.maximum(acc, 0.0), acc)
        return q2, jnp.dot(q2, w9, preferred_element_type=F32)

    Q2, p3 = _adjmm(adj_l, P2, [(W9, "full")], ep6, [256, 256])

    emb_gcn = embp[:, :32]
    h_enc = Z2[:, 256:288]
    emb_d = Q1[:, 64:72]
    z = jnp.concatenate([h_enc, emb_gcn, emb_d], axis=1)
    z_pad = jnp.pad(z, ((0, 0), (0, 56)))
    cl_pad = jnp.pad(cluster_layer, ((0, 6), (0, 56)))

    def ep7(acc, ex):
        zp, cl = ex
        z2s = jnp.sum(zp * zp, axis=1, keepdims=True)
        c2s = jnp.sum(cl * cl, axis=1)[None, :]
        cross = jnp.dot(zp, cl.T, preferred_element_type=F32)
        dist = z2s - 2.0 * cross + c2s
        qv = 1.0 / (1.0 + dist + 1e-8)
        qv = qv * qv / 2.0
        colid = lax.broadcasted_iota(jnp.int32, qv.shape, 1)
        qv = jnp.where(colid < 10, qv, 0.0)
        qn = qv / jnp.sum(qv, axis=1, keepdims=True)
        return acc, qn

    z_hat, qfull = _adjmm(adj_l, p3, [(z_pad, "i"), (cl_pad, "full")],
                          ep7, [256, 16])

    return (z_hat, qfull[:, :10], z, Z2[:, :256], Q2[:, 128:])


# speedup vs baseline: 14.6668x; 1.1028x over previous
import functools

import jax
import jax.numpy as jnp
from jax import lax
from jax.experimental import pallas as pl
from jax.experimental.pallas import tpu as pltpu
from jax.experimental.pallas import tpu_sc as plsc

N = 4096
E = 131072
THR = 0.1
BM = 512
NB = N // BM
F32 = jnp.float32


NTILES = 16
EPT = E // NTILES
NCHUNKS = 16
CROWS = N // NCHUNKS
CWORDS = CROWS * N
DUMPW = 2048
ZN = CWORDS // NTILES
ZB = 8192
IDX_ROWS = EPT // 128


def _count_body(edges_hbm, out_hbm, row_v, col_v, idx_a, idx_b, ones_v, zer_v,
                buf_sh, sem):
    cid = lax.axis_index("c")
    sid = lax.axis_index("s")

    ebase = pl.multiple_of(sid * EPT, 8)
    cp_r = pltpu.async_copy(edges_hbm.at[0, pl.ds(ebase, EPT)], row_v, sem)
    cp_c = pltpu.async_copy(edges_hbm.at[1, pl.ds(ebase, EPT)], col_v, sem)

    for j in range(8):
        ones_v[pl.ds(j * 16, 16)] = jnp.ones((16,), F32)

    def zbody(i, _):
        zer_v[pl.ds(i * 16, 16)] = jnp.zeros((16,), F32)
        return 0

    lax.fori_loop(0, ZB // 16, zbody, 0)
    cp_r.wait()
    cp_c.wait()

    lane = lax.iota(jnp.int32, 16)

    def compute_idx(idx_v, base):
        def ibody(i, _):
            for j in range(8):
                e0 = i * 128 + j * 16
                r = row_v[pl.ds(e0, 16)]
                c = col_v[pl.ds(e0, 16)]
                full = r * N + c - base
                valid = (full >= 0) & (full < CWORDS)
                dump = CWORDS + ((e0 + lane) & (DUMPW - 1))
                idx_v[i, pl.ds(j * 16, 16)] = jnp.where(valid, full, dump)
            return 0

        lax.fori_loop(0, IDX_ROWS, ibody, 0)

    def chunk_base(ch):
        return pl.multiple_of((cid * (NCHUNKS // 2) + ch) * CWORDS, 8)

    bufs = [idx_a, idx_b]
    compute_idx(bufs[0], chunk_base(0))
    out_cp = None

    for ch in range(NCHUNKS // 2):
        base = chunk_base(ch)
        if out_cp is not None:
            out_cp.wait()

        zcps = [
            pltpu.async_copy(
                zer_v,
                buf_sh.at[pl.ds(pl.multiple_of(sid * ZN + zz * ZB, 8), ZB)],
                sem)
            for zz in range(ZN // ZB)]

        @pl.when(sid == 0)
        def _zd():
            pltpu.sync_copy(zer_v.at[pl.ds(0, DUMPW)],
                            buf_sh.at[pl.ds(CWORDS, DUMPW)])

        for cp in zcps:
            cp.wait()

        plsc.subcore_barrier()

        idx_v = bufs[ch % 2]
        scps = [pltpu.async_copy(ones_v, buf_sh.at[idx_v.at[jj]], sem,
                                 add=True)
                for jj in range(IDX_ROWS)]
        if ch + 1 < NCHUNKS // 2:
            compute_idx(bufs[(ch + 1) % 2], chunk_base(ch + 1))
        for cp in scps:
            cp.wait()

        plsc.subcore_barrier()

        dst = pl.multiple_of(base + sid * ZN, 8)
        out_cp = pltpu.async_copy(
            buf_sh.at[pl.ds(pl.multiple_of(sid * ZN, 8), ZN)],
            out_hbm.at[pl.ds(dst, ZN)], sem)

    out_cp.wait()


def _build_count(edge_index):
    mesh = plsc.VectorSubcoreMesh(core_axis_name="c", subcore_axis_name="s")
    fn = pl.kernel(
        _count_body,
        out_type=jax.ShapeDtypeStruct((N * N,), F32),
        mesh=mesh,
        scratch_types=[
            pltpu.VMEM((EPT,), jnp.int32),
            pltpu.VMEM((EPT,), jnp.int32),
            pltpu.VMEM((IDX_ROWS, 128), jnp.int32),
            pltpu.VMEM((IDX_ROWS, 128), jnp.int32),
            pltpu.VMEM((128,), F32),
            pltpu.VMEM((ZB,), F32),
            pltpu.VMEM_SHARED((CWORDS + DUMPW,), F32),
            pltpu.SemaphoreType.DMA,
        ],
    )
    return fn(edge_index)


def _rownorm_body(x_ref, o_ref):
    x = x_ref[...]
    nrm = jnp.maximum(jnp.sqrt(jnp.sum(x * x, axis=1, keepdims=True)), 1e-8)
    o_ref[...] = x / nrm


def _rownorm(x):
    n, d = x.shape
    return pl.pallas_call(
        _rownorm_body,
        grid=(n // BM,),
        in_specs=[pl.BlockSpec((BM, d), lambda i: (i, 0))],
        out_specs=pl.BlockSpec((BM, d), lambda i: (i, 0)),
        out_shape=jax.ShapeDtypeStruct((n, d), F32),
    )(x)


def _simadj_body(*refs, with_prev, with_h, emit_cbf):
    idx = 0
    xi = refs[idx]; idx += 1
    xj = refs[idx]; idx += 1
    cref = refs[idx]; idx += 1
    if with_prev:
        a1_ref = refs[idx]; idx += 1
    if with_h:
        xr = refs[idx]; idx += 1
        wr = refs[idx]; idx += 1
    a_ref = refs[idx]; idx += 1
    dinv_ref = refs[idx]; idx += 1
    if emit_cbf:
        cbf_ref = refs[idx]; idx += 1
    if with_h:
        hpack_ref = refs[idx]; idx += 1
    acc_ref = refs[idx]

    i = pl.program_id(0)
    j = pl.program_id(1)
    s = jnp.dot(xi[...], xj[...].T, preferred_element_type=F32)
    ri = lax.broadcasted_iota(jnp.int32, (BM, BM), 0) + (i - j) * BM
    ci = lax.broadcasted_iota(jnp.int32, (BM, BM), 1)
    m = (s >= THR) & (ri != ci)
    if with_prev:
        m = m & (a1_ref[...] > 0)
    c = cref[...]
    if c.dtype != F32:
        c = c.astype(F32)
    a = jnp.where(m, s, 0.0) * c
    a_ref[...] = a.astype(jnp.bfloat16)
    if emit_cbf:
        cbf_ref[...] = c.astype(jnp.bfloat16)

    @pl.when(j == 0)
    def _z():
        acc_ref[...] = jnp.zeros_like(acc_ref)

    acc_ref[...] += jnp.sum(a, axis=1, keepdims=True)

    @pl.when(j == NB - 1)
    def _f():
        dinv_ref[...] = lax.rsqrt(acc_ref[...] + 1.0)
        if with_h:
            hpack_ref[...] = jnp.dot(xr[...], wr[...],
                                     preferred_element_type=F32)


def _simadj(xn, c, a1=None, hx=None, wcat=None, emit_cbf=False):
    with_prev = a1 is not None
    with_h = hx is not None
    args = [xn, xn, c]
    in_specs = [pl.BlockSpec((BM, xn.shape[1]), lambda i, j: (i, 0)),
                pl.BlockSpec((BM, xn.shape[1]), lambda i, j: (j, 0)),
                pl.BlockSpec((BM, BM), lambda i, j: (i, j))]
    if with_prev:
        args.append(a1)
        in_specs.append(pl.BlockSpec((BM, BM), lambda i, j: (i, j)))
    out_shapes = [jax.ShapeDtypeStruct((N, N), jnp.bfloat16),
                  jax.ShapeDtypeStruct((N, 1), F32)]
    out_specs = [pl.BlockSpec((BM, BM), lambda i, j: (i, j)),
                 pl.BlockSpec((BM, 1), lambda i, j: (i, 0))]
    if emit_cbf:
        out_shapes.append(jax.ShapeDtypeStruct((N, N), jnp.bfloat16))
        out_specs.append(pl.BlockSpec((BM, BM), lambda i, j: (i, j)))
    if with_h:
        args += [hx, wcat]
        in_specs += [pl.BlockSpec((BM, hx.shape[1]), lambda i, j: (i, 0)),
                     pl.BlockSpec(wcat.shape, lambda i, j: (0, 0))]
        out_shapes.append(jax.ShapeDtypeStruct((N, wcat.shape[1]), F32))
        out_specs.append(pl.BlockSpec((BM, wcat.shape[1]), lambda i, j: (i, 0)))
    return pl.pallas_call(
        functools.partial(_simadj_body, with_prev=with_prev, with_h=with_h,
                          emit_cbf=emit_cbf),
        grid=(NB, NB),
        in_specs=in_specs,
        out_specs=out_specs,
        out_shape=out_shapes,
        scratch_shapes=[pltpu.VMEM((BM, 1), F32)],
    )(*args)


def _adjmm_body(*refs, n_extra, n_out, epilogue, bscale, emit_pbf):
    idx = 0
    p_ref = refs[idx]; idx += 1
    b_ref = refs[idx]; idx += 1
    if bscale:
        bs_ref = refs[idx]; idx += 1
    extra_refs = refs[idx:idx + n_extra]; idx += n_extra
    out_refs = refs[idx:idx + n_out]; idx += n_out
    if emit_pbf:
        pbf_ref = refs[idx]; idx += 1
    acc_ref = refs[idx]

    k = pl.program_id(1)

    @pl.when(k == 0)
    def _z():
        acc_ref[...] = jnp.zeros_like(acc_ref)

    bb = b_ref[...]
    if bscale:
        bb = bs_ref[...] * bb
    pp = p_ref[...]
    if pp.dtype != jnp.bfloat16:
        pp = pp.astype(jnp.bfloat16)
        if emit_pbf:
            pbf_ref[...] = pp
    acc_ref[...] += jnp.dot(pp, bb.astype(jnp.bfloat16),
                            preferred_element_type=F32)

    @pl.when(k == NB - 1)
    def _f():
        outs = epilogue(acc_ref[...], [er[...] for er in extra_refs])
        for o_ref, o in zip(out_refs, outs):
            o_ref[...] = o


def _adjmm(p, b, extras, epilogue, out_widths, bscale=None, emit_pbf=False):
    w = b.shape[1]
    args = [p, b]
    in_specs = [pl.BlockSpec((BM, BM), lambda i, k: (i, k)),
                pl.BlockSpec((BM, w), lambda i, k: (k, 0))]
    if bscale is not None:
        args.append(bscale)
        in_specs.append(pl.BlockSpec((BM, 1), lambda i, k: (k, 0)))
    for arr, mode in extras:
        args.append(arr)
        if mode == "i":
            in_specs.append(pl.BlockSpec((BM, arr.shape[1]),
                                         lambda i, k: (i, 0)))
        else:
            in_specs.append(pl.BlockSpec(arr.shape, lambda i, k: (0, 0)))
    out_shapes = [jax.ShapeDtypeStruct((N, ow), F32) for ow in out_widths]
    out_specs = [pl.BlockSpec((BM, ow), lambda i, k: (i, 0))
                 for ow in out_widths]
    if emit_pbf:
        out_shapes.append(jax.ShapeDtypeStruct((N, N), jnp.bfloat16))
        out_specs.append(pl.BlockSpec((BM, BM), lambda i, k: (i, k)))
    res = pl.pallas_call(
        functools.partial(_adjmm_body, n_extra=len(extras),
                          n_out=len(out_widths), epilogue=epilogue,
                          bscale=bscale is not None, emit_pbf=emit_pbf),
        grid=(NB, NB),
        in_specs=in_specs,
        out_specs=out_specs,
        out_shape=out_shapes,
        scratch_shapes=[pltpu.VMEM((BM, w), F32)],
    )(*args)
    return res


def kernel(x, adj_l, dec, edge_index, Wse1, bse1, Wse2, bse2, Wsd1, bsd1,
           Wsd2, bsd2, We1, be1, We2, be2, Wg3, Wg4, Wg5, weight1, weight2,
           cluster_layer):
    C = _build_count(edge_index).reshape(N, N)
    xn = _rownorm(x)

    wcat = jnp.concatenate([Wse1, We1], axis=1)
    A1, dinv1, C_bf, hpack = _simadj(xn, C, hx=x, wcat=wcat, emit_cbf=True)
    h = hpack[:, :256]
    u1 = hpack[:, 256:]

    Wse2p = jnp.pad(Wse2, ((0, 0), (0, 96)))
    bse1r = bse1.reshape(1, -1)

    def ep1(acc, ex):
        dinv, hh, bb, w2p = ex
        h1 = jnp.maximum(dinv * acc + dinv * dinv * hh + bb, 0.0)
        nrm = jnp.maximum(jnp.sqrt(jnp.sum(h1 * h1, axis=1, keepdims=True)),
                          1e-8)
        return h1 / nrm, jnp.dot(h1, w2p, preferred_element_type=F32)

    h1n, h2p = _adjmm(A1, h,
                      [(dinv1, "i"), (h, "i"), (bse1r, "full"),
                       (Wse2p, "full")],
                      ep1, [256, 128], bscale=dinv1)

    A2, dinv2 = _simadj(h1n, C_bf, a1=A1)
    bse2p = jnp.pad(bse2, (0, 96)).reshape(1, -1)
    Wsd1p = jnp.pad(Wsd1, ((0, 96), (0, 0)))

    def ep2(acc, ex):
        dinv, hh, bb, wsd1p = ex
        e = dinv * acc + dinv * dinv * hh + bb
        return e, jnp.dot(e, wsd1p, preferred_element_type=F32)

    embp, g1 = _adjmm(A2, h2p,
                      [(dinv2, "i"), (h2p, "i"), (bse2p, "full"),
                       (Wsd1p, "full")],
                      ep2, [128, 256], bscale=dinv2)

    B1 = jnp.concatenate([g1, u1], axis=1)
    bias3 = jnp.concatenate([bsd1, be1]).reshape(1, -1)
    W6 = jnp.zeros((384, 384), F32)
    W6 = W6.at[:256, :256].set(Wsd2)
    W6 = W6.at[256:, 256:288].set(We2)

    def ep3(acc, ex):
        bb, w6 = ex
        t = jnp.maximum(acc + bb, 0.0)
        return (jnp.dot(t, w6, preferred_element_type=F32),)

    (B2,) = _adjmm(C_bf, B1, [(bias3, "full"), (W6, "full")], ep3, [384])

    bias4 = jnp.concatenate([bsd2, be2, jnp.zeros((96,), F32)]).reshape(1, -1)
    W7z = jnp.zeros((384, 128), F32).at[256:288, 0:64].set(Wg3)
    W7d = jnp.zeros((128, 128), F32).at[:, 64:72].set(weight1)

    def ep4(acc, ex):
        bb, w7z, w7d, dcb = ex
        z2 = acc + bb
        p1 = (jnp.dot(z2, w7z, preferred_element_type=F32)
              + jnp.dot(dcb, w7d, preferred_element_type=F32))
        return z2, p1

    Z2, P1 = _adjmm(C_bf, B2,
                    [(bias4, "full"), (W7z, "full"), (W7d, "full"),
                     (dec, "i")],
                    ep4, [384, 128])

    W8 = jnp.zeros((128, 256), F32)
    W8 = W8.at[0:64, 0:128].set(Wg4)
    W8 = W8.at[64:72, 128:256].set(weight2)

    def ep5(acc, ex):
        (w8,) = ex
        colid = lax.broadcasted_iota(jnp.int32, acc.shape, 1)
        q1 = jnp.where(colid < 64, jnp.maximum(acc, 0.0), acc)
        return q1, jnp.dot(q1, w8, preferred_element_type=F32)

    Q1, P2, adj_bf = _adjmm(adj_l, P1, [(W8, "full")], ep5, [128, 256],
                            emit_pbf=True)

    W9 = jnp.zeros((256, 256), F32).at[0:128, :].set(Wg5)

    def ep6(acc, ex):
        (w9,) = ex
        colid = lax.broadcasted_iota(jnp.int32, acc.shape, 1)
        q2 = jnp.where(colid < 128, jnp.maximum(acc, 0.0), acc)
        return q2, jnp.dot(q2, w9, preferred_element_type=F32)

    Q2, p3 = _adjmm(adj_bf, P2, [(W9, "full")], ep6, [256, 256])

    emb_gcn = embp[:, :32]
    h_enc = Z2[:, 256:288]
    emb_d = Q1[:, 64:72]
    z = jnp.concatenate([h_enc, emb_gcn, emb_d], axis=1)
    z_pad = jnp.pad(z, ((0, 0), (0, 56)))
    cl_pad = jnp.pad(cluster_layer, ((0, 6), (0, 56)))

    def ep7(acc, ex):
        zp, cl = ex
        z2s = jnp.sum(zp * zp, axis=1, keepdims=True)
        c2s = jnp.sum(cl * cl, axis=1)[None, :]
        cross = jnp.dot(zp, cl.T, preferred_element_type=F32)
        dist = z2s - 2.0 * cross + c2s
        qv = 1.0 / (1.0 + dist + 1e-8)
        qv = qv * qv / 2.0
        colid = lax.broadcasted_iota(jnp.int32, qv.shape, 1)
        qv = jnp.where(colid < 10, qv, 0.0)
        qn = qv / jnp.sum(qv, axis=1, keepdims=True)
        return acc, qn

    z_hat, qfull = _adjmm(adj_bf, p3, [(z_pad, "i"), (cl_pad, "full")],
                          ep7, [256, 16])

    return (z_hat, qfull[:, :10], z, Z2[:, :256], Q2[:, 128:])


# speedup vs baseline: 22.5290x; 1.5361x over previous
import functools

import jax
import jax.numpy as jnp
from jax import lax
from jax.experimental import pallas as pl
from jax.experimental.pallas import tpu as pltpu
from jax.experimental.pallas import tpu_sc as plsc

N = 4096
E = 131072
THR = 0.1
BM = 1024
NB = N // BM
F32 = jnp.float32


NTILES = 16
EPT = E // NTILES
NCHUNKS = 16
CROWS = N // NCHUNKS
CWORDS = CROWS * N
DUMPW = 2048
ZN = CWORDS // NTILES
ZB = 8192
IDX_ROWS = EPT // 128


def _count_body(edges_hbm, out_hbm, row_v, col_v, idx_a, idx_b, ones_v, zer_v,
                buf_sh, sem):
    cid = lax.axis_index("c")
    sid = lax.axis_index("s")

    ebase = pl.multiple_of(sid * EPT, 8)
    cp_r = pltpu.async_copy(edges_hbm.at[0, pl.ds(ebase, EPT)], row_v, sem)
    cp_c = pltpu.async_copy(edges_hbm.at[1, pl.ds(ebase, EPT)], col_v, sem)

    for j in range(8):
        ones_v[pl.ds(j * 16, 16)] = jnp.ones((16,), F32)

    def zbody(i, _):
        zer_v[pl.ds(i * 16, 16)] = jnp.zeros((16,), F32)
        return 0

    lax.fori_loop(0, ZB // 16, zbody, 0)
    cp_r.wait()
    cp_c.wait()

    lane = lax.iota(jnp.int32, 16)

    def compute_idx(idx_v, base):
        def ibody(i, _):
            for j in range(8):
                e0 = i * 128 + j * 16
                r = row_v[pl.ds(e0, 16)]
                c = col_v[pl.ds(e0, 16)]
                full = r * N + c - base
                valid = (full >= 0) & (full < CWORDS)
                dump = CWORDS + ((e0 + lane) & (DUMPW - 1))
                idx_v[i, pl.ds(j * 16, 16)] = jnp.where(valid, full, dump)
            return 0

        lax.fori_loop(0, IDX_ROWS, ibody, 0)

    def chunk_base(ch):
        return pl.multiple_of((cid * (NCHUNKS // 2) + ch) * CWORDS, 8)

    bufs = [idx_a, idx_b]
    compute_idx(bufs[0], chunk_base(0))
    out_cp = None

    for ch in range(NCHUNKS // 2):
        base = chunk_base(ch)
        if out_cp is not None:
            out_cp.wait()

        zcps = [
            pltpu.async_copy(
                zer_v,
                buf_sh.at[pl.ds(pl.multiple_of(sid * ZN + zz * ZB, 8), ZB)],
                sem)
            for zz in range(ZN // ZB)]

        @pl.when(sid == 0)
        def _zd():
            pltpu.sync_copy(zer_v.at[pl.ds(0, DUMPW)],
                            buf_sh.at[pl.ds(CWORDS, DUMPW)])

        for cp in zcps:
            cp.wait()

        plsc.subcore_barrier()

        idx_v = bufs[ch % 2]
        scps = [pltpu.async_copy(ones_v, buf_sh.at[idx_v.at[jj]], sem,
                                 add=True)
                for jj in range(IDX_ROWS)]
        if ch + 1 < NCHUNKS // 2:
            compute_idx(bufs[(ch + 1) % 2], chunk_base(ch + 1))
        for cp in scps:
            cp.wait()

        plsc.subcore_barrier()

        dst = pl.multiple_of(base + sid * ZN, 8)
        out_cp = pltpu.async_copy(
            buf_sh.at[pl.ds(pl.multiple_of(sid * ZN, 8), ZN)],
            out_hbm.at[pl.ds(dst, ZN)], sem)

    out_cp.wait()


def _build_count(edge_index):
    mesh = plsc.VectorSubcoreMesh(core_axis_name="c", subcore_axis_name="s")
    fn = pl.kernel(
        _count_body,
        out_type=jax.ShapeDtypeStruct((N * N,), F32),
        mesh=mesh,
        scratch_types=[
            pltpu.VMEM((EPT,), jnp.int32),
            pltpu.VMEM((EPT,), jnp.int32),
            pltpu.VMEM((IDX_ROWS, 128), jnp.int32),
            pltpu.VMEM((IDX_ROWS, 128), jnp.int32),
            pltpu.VMEM((128,), F32),
            pltpu.VMEM((ZB,), F32),
            pltpu.VMEM_SHARED((CWORDS + DUMPW,), F32),
            pltpu.SemaphoreType.DMA,
        ],
    )
    return fn(edge_index)


def _rownorm_body(x_ref, o_ref):
    x = x_ref[...]
    nrm = jnp.maximum(jnp.sqrt(jnp.sum(x * x, axis=1, keepdims=True)), 1e-8)
    o_ref[...] = x / nrm


def _rownorm(x):
    n, d = x.shape
    return pl.pallas_call(
        _rownorm_body,
        grid=(n // BM,),
        in_specs=[pl.BlockSpec((BM, d), lambda i: (i, 0))],
        out_specs=pl.BlockSpec((BM, d), lambda i: (i, 0)),
        out_shape=jax.ShapeDtypeStruct((n, d), F32),
    )(x)


def _simadj_body(*refs, with_prev, with_h, emit_cbf):
    idx = 0
    xi = refs[idx]; idx += 1
    xj = refs[idx]; idx += 1
    cref = refs[idx]; idx += 1
    if with_prev:
        a1_ref = refs[idx]; idx += 1
    if with_h:
        xr = refs[idx]; idx += 1
        wr = refs[idx]; idx += 1
    a_ref = refs[idx]; idx += 1
    dinv_ref = refs[idx]; idx += 1
    if emit_cbf:
        cbf_ref = refs[idx]; idx += 1
    if with_h:
        hpack_ref = refs[idx]; idx += 1
    acc_ref = refs[idx]

    i = pl.program_id(0)
    j = pl.program_id(1)
    s = jnp.dot(xi[...], xj[...].T, preferred_element_type=F32)
    ri = lax.broadcasted_iota(jnp.int32, (BM, BM), 0) + (i - j) * BM
    ci = lax.broadcasted_iota(jnp.int32, (BM, BM), 1)
    m = (s >= THR) & (ri != ci)
    if with_prev:
        m = m & (a1_ref[...] > 0)
    c = cref[...]
    if c.dtype != F32:
        c = c.astype(F32)
    a = jnp.where(m, s, 0.0) * c
    a_ref[...] = a.astype(jnp.bfloat16)
    if emit_cbf:
        cbf_ref[...] = c.astype(jnp.bfloat16)

    @pl.when(j == 0)
    def _z():
        acc_ref[...] = jnp.zeros_like(acc_ref)

    acc_ref[...] += jnp.sum(a, axis=1, keepdims=True)

    @pl.when(j == NB - 1)
    def _f():
        dinv_ref[...] = lax.rsqrt(acc_ref[...] + 1.0)
        if with_h:
            hpack_ref[...] = jnp.dot(xr[...], wr[...],
                                     preferred_element_type=F32)


def _simadj(xn, c, a1=None, hx=None, wcat=None, emit_cbf=False):
    with_prev = a1 is not None
    with_h = hx is not None
    args = [xn, xn, c]
    in_specs = [pl.BlockSpec((BM, xn.shape[1]), lambda i, j: (i, 0)),
                pl.BlockSpec((BM, xn.shape[1]), lambda i, j: (j, 0)),
                pl.BlockSpec((BM, BM), lambda i, j: (i, j))]
    if with_prev:
        args.append(a1)
        in_specs.append(pl.BlockSpec((BM, BM), lambda i, j: (i, j)))
    out_shapes = [jax.ShapeDtypeStruct((N, N), jnp.bfloat16),
                  jax.ShapeDtypeStruct((N, 1), F32)]
    out_specs = [pl.BlockSpec((BM, BM), lambda i, j: (i, j)),
                 pl.BlockSpec((BM, 1), lambda i, j: (i, 0))]
    if emit_cbf:
        out_shapes.append(jax.ShapeDtypeStruct((N, N), jnp.bfloat16))
        out_specs.append(pl.BlockSpec((BM, BM), lambda i, j: (i, j)))
    if with_h:
        args += [hx, wcat]
        in_specs += [pl.BlockSpec((BM, hx.shape[1]), lambda i, j: (i, 0)),
                     pl.BlockSpec(wcat.shape, lambda i, j: (0, 0))]
        out_shapes.append(jax.ShapeDtypeStruct((N, wcat.shape[1]), F32))
        out_specs.append(pl.BlockSpec((BM, wcat.shape[1]), lambda i, j: (i, 0)))
    return pl.pallas_call(
        functools.partial(_simadj_body, with_prev=with_prev, with_h=with_h,
                          emit_cbf=emit_cbf),
        grid=(NB, NB),
        in_specs=in_specs,
        out_specs=out_specs,
        out_shape=out_shapes,
        scratch_shapes=[pltpu.VMEM((BM, 1), F32)],
    )(*args)


def _adjmm_body(*refs, n_extra, n_out, epilogue, bscale, emit_pbf):
    idx = 0
    p_ref = refs[idx]; idx += 1
    b_ref = refs[idx]; idx += 1
    if bscale:
        bs_ref = refs[idx]; idx += 1
    extra_refs = refs[idx:idx + n_extra]; idx += n_extra
    out_refs = refs[idx:idx + n_out]; idx += n_out
    if emit_pbf:
        pbf_ref = refs[idx]; idx += 1
    acc_ref = refs[idx]

    k = pl.program_id(1)

    @pl.when(k == 0)
    def _z():
        acc_ref[...] = jnp.zeros_like(acc_ref)

    bb = b_ref[...]
    if bscale:
        bb = bs_ref[...] * bb
    pp = p_ref[...]
    if pp.dtype != jnp.bfloat16:
        pp = pp.astype(jnp.bfloat16)
        if emit_pbf:
            pbf_ref[...] = pp
    acc_ref[...] += jnp.dot(pp, bb.astype(jnp.bfloat16),
                            preferred_element_type=F32)

    @pl.when(k == NB - 1)
    def _f():
        outs = epilogue(acc_ref[...], [er[...] for er in extra_refs])
        for o_ref, o in zip(out_refs, outs):
            o_ref[...] = o


def _adjmm(p, b, extras, epilogue, out_widths, bscale=None, emit_pbf=False):
    w = b.shape[1]
    args = [p, b]
    in_specs = [pl.BlockSpec((BM, BM), lambda i, k: (i, k)),
                pl.BlockSpec((BM, w), lambda i, k: (k, 0))]
    if bscale is not None:
        args.append(bscale)
        in_specs.append(pl.BlockSpec((BM, 1), lambda i, k: (k, 0)))
    for arr, mode in extras:
        args.append(arr)
        if mode == "i":
            in_specs.append(pl.BlockSpec((BM, arr.shape[1]),
                                         lambda i, k: (i, 0)))
        else:
            in_specs.append(pl.BlockSpec(arr.shape, lambda i, k: (0, 0)))
    out_shapes = [jax.ShapeDtypeStruct((N, ow), F32) for ow in out_widths]
    out_specs = [pl.BlockSpec((BM, ow), lambda i, k: (i, 0))
                 for ow in out_widths]
    if emit_pbf:
        out_shapes.append(jax.ShapeDtypeStruct((N, N), jnp.bfloat16))
        out_specs.append(pl.BlockSpec((BM, BM), lambda i, k: (i, k)))
    res = pl.pallas_call(
        functools.partial(_adjmm_body, n_extra=len(extras),
                          n_out=len(out_widths), epilogue=epilogue,
                          bscale=bscale is not None, emit_pbf=emit_pbf),
        grid=(NB, NB),
        in_specs=in_specs,
        out_specs=out_specs,
        out_shape=out_shapes,
        scratch_shapes=[pltpu.VMEM((BM, w), F32)],
    )(*args)
    return res


def kernel(x, adj_l, dec, edge_index, Wse1, bse1, Wse2, bse2, Wsd1, bsd1,
           Wsd2, bsd2, We1, be1, We2, be2, Wg3, Wg4, Wg5, weight1, weight2,
           cluster_layer):
    C = _build_count(edge_index).reshape(N, N)
    xn = _rownorm(x)

    wcat = jnp.concatenate([Wse1, We1], axis=1)
    A1, dinv1, C_bf, hpack = _simadj(xn, C, hx=x, wcat=wcat, emit_cbf=True)
    h = hpack[:, :256]
    u1 = hpack[:, 256:]

    Wse2p = jnp.pad(Wse2, ((0, 0), (0, 96)))
    bse1r = bse1.reshape(1, -1)

    def ep1(acc, ex):
        dinv, hh, bb, w2p = ex
        h1 = jnp.maximum(dinv * acc + dinv * dinv * hh + bb, 0.0)
        nrm = jnp.maximum(jnp.sqrt(jnp.sum(h1 * h1, axis=1, keepdims=True)),
                          1e-8)
        return h1 / nrm, jnp.dot(h1, w2p, preferred_element_type=F32)

    h1n, h2p = _adjmm(A1, h,
                      [(dinv1, "i"), (h, "i"), (bse1r, "full"),
                       (Wse2p, "full")],
                      ep1, [256, 128], bscale=dinv1)

    A2, dinv2 = _simadj(h1n, C_bf, a1=A1)
    bse2p = jnp.pad(bse2, (0, 96)).reshape(1, -1)
    Wsd1p = jnp.pad(Wsd1, ((0, 96), (0, 0)))

    def ep2(acc, ex):
        dinv, hh, bb, wsd1p = ex
        e = dinv * acc + dinv * dinv * hh + bb
        return e, jnp.dot(e, wsd1p, preferred_element_type=F32)

    embp, g1 = _adjmm(A2, h2p,
                      [(dinv2, "i"), (h2p, "i"), (bse2p, "full"),
                       (Wsd1p, "full")],
                      ep2, [128, 256], bscale=dinv2)

    B1 = jnp.concatenate([g1, u1], axis=1)
    bias3 = jnp.concatenate([bsd1, be1]).reshape(1, -1)
    W6 = jnp.zeros((384, 384), F32)
    W6 = W6.at[:256, :256].set(Wsd2)
    W6 = W6.at[256:, 256:288].set(We2)

    def ep3(acc, ex):
        bb, w6 = ex
        t = jnp.maximum(acc + bb, 0.0)
        return (jnp.dot(t, w6, preferred_element_type=F32),)

    (B2,) = _adjmm(C_bf, B1, [(bias3, "full"), (W6, "full")], ep3, [384])

    bias4 = jnp.concatenate([bsd2, be2, jnp.zeros((96,), F32)]).reshape(1, -1)
    W7z = jnp.zeros((384, 128), F32).at[256:288, 0:64].set(Wg3)
    W7d = jnp.zeros((128, 128), F32).at[:, 64:72].set(weight1)

    def ep4(acc, ex):
        bb, w7z, w7d, dcb = ex
        z2 = acc + bb
        p1 = (jnp.dot(z2, w7z, preferred_element_type=F32)
              + jnp.dot(dcb, w7d, preferred_element_type=F32))
        return z2, p1

    Z2, P1 = _adjmm(C_bf, B2,
                    [(bias4, "full"), (W7z, "full"), (W7d, "full"),
                     (dec, "i")],
                    ep4, [384, 128])

    W8 = jnp.zeros((128, 256), F32)
    W8 = W8.at[0:64, 0:128].set(Wg4)
    W8 = W8.at[64:72, 128:256].set(weight2)

    def ep5(acc, ex):
        (w8,) = ex
        colid = lax.broadcasted_iota(jnp.int32, acc.shape, 1)
        q1 = jnp.where(colid < 64, jnp.maximum(acc, 0.0), acc)
        return q1, jnp.dot(q1, w8, preferred_element_type=F32)

    Q1, P2, adj_bf = _adjmm(adj_l, P1, [(W8, "full")], ep5, [128, 256],
                            emit_pbf=True)

    W9 = jnp.zeros((256, 256), F32).at[0:128, :].set(Wg5)

    def ep6(acc, ex):
        (w9,) = ex
        colid = lax.broadcasted_iota(jnp.int32, acc.shape, 1)
        q2 = jnp.where(colid < 128, jnp.maximum(acc, 0.0), acc)
        return q2, jnp.dot(q2, w9, preferred_element_type=F32)

    Q2, p3 = _adjmm(adj_bf, P2, [(W9, "full")], ep6, [256, 256])

    emb_gcn = embp[:, :32]
    h_enc = Z2[:, 256:288]
    emb_d = Q1[:, 64:72]
    z = jnp.concatenate([h_enc, emb_gcn, emb_d], axis=1)
    z_pad = jnp.pad(z, ((0, 0), (0, 56)))
    cl_pad = jnp.pad(cluster_layer, ((0, 6), (0, 56)))

    def ep7(acc, ex):
        zp, cl = ex
        z2s = jnp.sum(zp * zp, axis=1, keepdims=True)
        c2s = jnp.sum(cl * cl, axis=1)[None, :]
        cross = jnp.dot(zp, cl.T, preferred_element_type=F32)
        dist = z2s - 2.0 * cross + c2s
        qv = 1.0 / (1.0 + dist + 1e-8)
        qv = qv * qv / 2.0
        colid = lax.broadcasted_iota(jnp.int32, qv.shape, 1)
        qv = jnp.where(colid < 10, qv, 0.0)
        qn = qv / jnp.sum(qv, axis=1, keepdims=True)
        return acc, qn

    z_hat, qfull = _adjmm(adj_bf, p3, [(z_pad, "i"), (cl_pad, "full")],
                          ep7, [256, 16])

    return (z_hat, qfull[:, :10], z, Z2[:, :256], Q2[:, 128:])


# speedup vs baseline: 25.4785x; 1.1309x over previous
import functools

import jax
import jax.numpy as jnp
from jax import lax
from jax.experimental import pallas as pl
from jax.experimental.pallas import tpu as pltpu
from jax.experimental.pallas import tpu_sc as plsc

N = 4096
E = 131072
THR = 0.1
BM = 1024
NB = N // BM
F32 = jnp.float32


NTILES = 16
EPT = E // NTILES
NCHUNKS = 16
CROWS = N // NCHUNKS
CWORDS = CROWS * N
DUMPW = 2048
ZN = CWORDS // NTILES
ZROWS = CROWS // NTILES
ZB = 8192
IDX_ROWS = EPT // 128
BF16 = jnp.bfloat16


def _count_body(edges_hbm, out_hbm, row_v, col_v, idx_a, idx_b, ones_v, zer_v,
                buf_sh, sem):
    cid = lax.axis_index("c")
    sid = lax.axis_index("s")

    ebase = pl.multiple_of(sid * EPT, 8)
    cp_r = pltpu.async_copy(edges_hbm.at[0, pl.ds(ebase, EPT)], row_v, sem)
    cp_c = pltpu.async_copy(edges_hbm.at[1, pl.ds(ebase, EPT)], col_v, sem)

    for j in range(8):
        ones_v[pl.ds(j * 16, 16)] = jnp.ones((16,), F32)

    def zbody(i, _):
        zer_v[pl.ds(i * 16, 16)] = jnp.zeros((16,), F32)
        return 0

    lax.fori_loop(0, ZB // 16, zbody, 0)
    cp_r.wait()
    cp_c.wait()

    lane = lax.iota(jnp.int32, 16)

    def compute_idx(idx_v, base):
        def ibody(i, _):
            for j in range(8):
                e0 = i * 128 + j * 16
                r = row_v[pl.ds(e0, 16)]
                c = col_v[pl.ds(e0, 16)]
                full = r * N + c - base
                valid = (full >= 0) & (full < CWORDS)
                dump = CWORDS + ((e0 + lane) & (DUMPW - 1))
                idx_v[i, pl.ds(j * 16, 16)] = jnp.where(valid, full, dump)
            return 0

        lax.fori_loop(0, IDX_ROWS, ibody, 0)

    def chunk_base(ch):
        return pl.multiple_of((cid * (NCHUNKS // 2) + ch) * CWORDS, 8)

    bufs = [idx_a, idx_b]
    compute_idx(bufs[0], chunk_base(0))
    out_cps = []

    for ch in range(NCHUNKS // 2):
        base = chunk_base(ch)
        for cp in out_cps:
            cp.wait()
        out_cps = []

        zcps = [
            pltpu.async_copy(
                zer_v,
                buf_sh.at[pl.ds(pl.multiple_of(sid * ZN + zz * ZB, 8), ZB)],
                sem)
            for zz in range(ZN // ZB)]

        @pl.when(sid == 0)
        def _zd():
            pltpu.sync_copy(zer_v.at[pl.ds(0, DUMPW)],
                            buf_sh.at[pl.ds(CWORDS, DUMPW)])

        for cp in zcps:
            cp.wait()

        plsc.subcore_barrier()

        idx_v = bufs[ch % 2]
        scps = [pltpu.async_copy(ones_v, buf_sh.at[idx_v.at[jj]], sem,
                                 add=True)
                for jj in range(IDX_ROWS)]
        if ch + 1 < NCHUNKS // 2:
            compute_idx(bufs[(ch + 1) % 2], chunk_base(ch + 1))
        for cp in scps:
            cp.wait()

        plsc.subcore_barrier()

        dst = pl.multiple_of(base + sid * ZN, 8)
        out_cps.append(pltpu.async_copy(
            buf_sh.at[pl.ds(pl.multiple_of(sid * ZN, 8), ZN)],
            out_hbm.at[pl.ds(dst, ZN)], sem))

    for cp in out_cps:
        cp.wait()


def _build_count(edge_index):
    mesh = plsc.VectorSubcoreMesh(core_axis_name="c", subcore_axis_name="s")
    fn = pl.kernel(
        _count_body,
        out_type=jax.ShapeDtypeStruct((N * N,), F32),
        mesh=mesh,
        scratch_types=[
            pltpu.VMEM((EPT,), jnp.int32),
            pltpu.VMEM((EPT,), jnp.int32),
            pltpu.VMEM((IDX_ROWS, 128), jnp.int32),
            pltpu.VMEM((IDX_ROWS, 128), jnp.int32),
            pltpu.VMEM((128,), F32),
            pltpu.VMEM((ZB,), F32),
            pltpu.VMEM_SHARED((CWORDS + DUMPW,), F32),
            pltpu.SemaphoreType.DMA,
        ],
    )
    return fn(edge_index)


def _retile_body(x_ref, o_ref):
    o_ref[...] = x_ref[...].reshape(o_ref.shape).astype(BF16)


def _retile(cflat):
    rb = 512
    return pl.pallas_call(
        _retile_body,
        grid=(N // rb,),
        in_specs=[pl.BlockSpec((rb * N,), lambda i: (i,))],
        out_specs=pl.BlockSpec((rb, N), lambda i: (i, 0)),
        out_shape=jax.ShapeDtypeStruct((N, N), BF16),
    )(cflat)


def _rownorm_body(x_ref, o_ref):
    x = x_ref[...]
    nrm = jnp.maximum(jnp.sqrt(jnp.sum(x * x, axis=1, keepdims=True)), 1e-8)
    o_ref[...] = x / nrm


def _rownorm(x):
    n, d = x.shape
    return pl.pallas_call(
        _rownorm_body,
        grid=(n // BM,),
        in_specs=[pl.BlockSpec((BM, d), lambda i: (i, 0))],
        out_specs=pl.BlockSpec((BM, d), lambda i: (i, 0)),
        out_shape=jax.ShapeDtypeStruct((n, d), F32),
    )(x)


def _simadj_body(*refs, with_prev, with_h, hsplit):
    idx = 0
    xi = refs[idx]; idx += 1
    xj = refs[idx]; idx += 1
    cref = refs[idx]; idx += 1
    if with_prev:
        a1_ref = refs[idx]; idx += 1
    if with_h:
        xr = refs[idx]; idx += 1
        wr = refs[idx]; idx += 1
    a_ref = refs[idx]; idx += 1
    dinv_ref = refs[idx]; idx += 1
    if with_h:
        h_ref = refs[idx]; idx += 1
        u_ref = refs[idx]; idx += 1
    acc_ref = refs[idx]

    i = pl.program_id(0)
    j = pl.program_id(1)
    s = jnp.dot(xi[...], xj[...].T, preferred_element_type=F32)
    ri = lax.broadcasted_iota(jnp.int32, (BM, BM), 0) + (i - j) * BM
    ci = lax.broadcasted_iota(jnp.int32, (BM, BM), 1)
    m = (s >= THR) & (ri != ci)
    if with_prev:
        m = m & (a1_ref[...] > 0)
    c = cref[...]
    if c.dtype != F32:
        c = c.astype(F32)
    a = jnp.where(m, s, 0.0) * c
    a_ref[...] = a.astype(jnp.bfloat16)

    @pl.when(j == 0)
    def _z():
        acc_ref[...] = jnp.zeros_like(acc_ref)

    acc_ref[...] += jnp.sum(a, axis=1, keepdims=True)

    @pl.when(j == NB - 1)
    def _f():
        dinv_ref[...] = lax.rsqrt(acc_ref[...] + 1.0)
        if with_h:
            hh = jnp.dot(xr[...], wr[...], preferred_element_type=F32)
            h_ref[...] = hh[:, :hsplit]
            u_ref[...] = hh[:, hsplit:]


def _simadj(xn, c, a1=None, hx=None, wcat=None, hsplit=256):
    with_prev = a1 is not None
    with_h = hx is not None
    args = [xn, xn, c]
    in_specs = [pl.BlockSpec((BM, xn.shape[1]), lambda i, j: (i, 0)),
                pl.BlockSpec((BM, xn.shape[1]), lambda i, j: (j, 0)),
                pl.BlockSpec((BM, BM), lambda i, j: (i, j))]
    if with_prev:
        args.append(a1)
        in_specs.append(pl.BlockSpec((BM, BM), lambda i, j: (i, j)))
    out_shapes = [jax.ShapeDtypeStruct((N, N), jnp.bfloat16),
                  jax.ShapeDtypeStruct((N, 1), F32)]
    out_specs = [pl.BlockSpec((BM, BM), lambda i, j: (i, j)),
                 pl.BlockSpec((BM, 1), lambda i, j: (i, 0))]
    if with_h:
        args += [hx, wcat]
        in_specs += [pl.BlockSpec((BM, hx.shape[1]), lambda i, j: (i, 0)),
                     pl.BlockSpec(wcat.shape, lambda i, j: (0, 0))]
        wtot = wcat.shape[1]
        out_shapes += [jax.ShapeDtypeStruct((N, hsplit), F32),
                       jax.ShapeDtypeStruct((N, wtot - hsplit), F32)]
        out_specs += [pl.BlockSpec((BM, hsplit), lambda i, j: (i, 0)),
                      pl.BlockSpec((BM, wtot - hsplit), lambda i, j: (i, 0))]
    return pl.pallas_call(
        functools.partial(_simadj_body, with_prev=with_prev, with_h=with_h,
                          hsplit=hsplit),
        grid=(NB, NB),
        in_specs=in_specs,
        out_specs=out_specs,
        out_shape=out_shapes,
        scratch_shapes=[pltpu.VMEM((BM, 1), F32)],
    )(*args)


def _adjmm_body(*refs, n_extra, n_out, epilogue, bscale, emit_pbf):
    idx = 0
    p_ref = refs[idx]; idx += 1
    b_ref = refs[idx]; idx += 1
    if bscale:
        bs_ref = refs[idx]; idx += 1
    extra_refs = refs[idx:idx + n_extra]; idx += n_extra
    out_refs = refs[idx:idx + n_out]; idx += n_out
    if emit_pbf:
        pbf_ref = refs[idx]; idx += 1
    acc_ref = refs[idx]

    k = pl.program_id(1)

    @pl.when(k == 0)
    def _z():
        acc_ref[...] = jnp.zeros_like(acc_ref)

    bb = b_ref[...]
    if bscale:
        bb = bs_ref[...] * bb
    pp = p_ref[...]
    if pp.dtype != jnp.bfloat16:
        pp = pp.astype(jnp.bfloat16)
        if emit_pbf:
            pbf_ref[...] = pp
    acc_ref[...] += jnp.dot(pp, bb.astype(jnp.bfloat16),
                            preferred_element_type=F32)

    @pl.when(k == NB - 1)
    def _f():
        outs = epilogue(acc_ref[...], [er[...] for er in extra_refs])
        for o_ref, o in zip(out_refs, outs):
            o_ref[...] = o


def _adjmm(p, b, extras, epilogue, out_widths, bscale=None, emit_pbf=False):
    w = b.shape[1]
    args = [p, b]
    in_specs = [pl.BlockSpec((BM, BM), lambda i, k: (i, k)),
                pl.BlockSpec((BM, w), lambda i, k: (k, 0))]
    if bscale is not None:
        args.append(bscale)
        in_specs.append(pl.BlockSpec((BM, 1), lambda i, k: (k, 0)))
    for arr, mode in extras:
        args.append(arr)
        if mode == "i":
            in_specs.append(pl.BlockSpec((BM, arr.shape[1]),
                                         lambda i, k: (i, 0)))
        else:
            in_specs.append(pl.BlockSpec(arr.shape, lambda i, k: (0, 0)))
    out_shapes = [jax.ShapeDtypeStruct((N, ow), F32) for ow in out_widths]
    out_specs = [pl.BlockSpec((BM, ow), lambda i, k: (i, 0))
                 for ow in out_widths]
    if emit_pbf:
        out_shapes.append(jax.ShapeDtypeStruct((N, N), jnp.bfloat16))
        out_specs.append(pl.BlockSpec((BM, BM), lambda i, k: (i, k)))
    res = pl.pallas_call(
        functools.partial(_adjmm_body, n_extra=len(extras),
                          n_out=len(out_widths), epilogue=epilogue,
                          bscale=bscale is not None, emit_pbf=emit_pbf),
        grid=(NB, NB),
        in_specs=in_specs,
        out_specs=out_specs,
        out_shape=out_shapes,
        scratch_shapes=[pltpu.VMEM((BM, w), F32)],
    )(*args)
    return res


def kernel(x, adj_l, dec, edge_index, Wse1, bse1, Wse2, bse2, Wsd1, bsd1,
           Wsd2, bsd2, We1, be1, We2, be2, Wg3, Wg4, Wg5, weight1, weight2,
           cluster_layer):
    C_bf = _retile(_build_count(edge_index))
    xn = _rownorm(x)

    wcat = jnp.concatenate([Wse1, We1], axis=1)
    A1, dinv1, h, u1 = _simadj(xn, C_bf, hx=x, wcat=wcat, hsplit=256)

    Wse2p = jnp.pad(Wse2, ((0, 0), (0, 96)))
    bse1r = bse1.reshape(1, -1)

    def ep1(acc, ex):
        dinv, hh, bb, w2p = ex
        h1 = jnp.maximum(dinv * acc + dinv * dinv * hh + bb, 0.0)
        nrm = jnp.maximum(jnp.sqrt(jnp.sum(h1 * h1, axis=1, keepdims=True)),
                          1e-8)
        return h1 / nrm, jnp.dot(h1, w2p, preferred_element_type=F32)

    h1n, h2p = _adjmm(A1, h,
                      [(dinv1, "i"), (h, "i"), (bse1r, "full"),
                       (Wse2p, "full")],
                      ep1, [256, 128], bscale=dinv1)

    A2, dinv2 = _simadj(h1n, C_bf, a1=A1)
    bse2p = jnp.pad(bse2, (0, 96)).reshape(1, -1)
    Wsd1p = jnp.pad(Wsd1, ((0, 96), (0, 0)))

    def ep2(acc, ex):
        dinv, hh, bb, wsd1p = ex
        e = dinv * acc + dinv * dinv * hh + bb
        return e[:, :32], jnp.dot(e, wsd1p, preferred_element_type=F32)

    emb32, g1 = _adjmm(A2, h2p,
                       [(dinv2, "i"), (h2p, "i"), (bse2p, "full"),
                        (Wsd1p, "full")],
                       ep2, [32, 256], bscale=dinv2)

    B1 = jnp.concatenate([g1, u1], axis=1)
    bias3 = jnp.concatenate([bsd1, be1]).reshape(1, -1)
    W6 = jnp.zeros((384, 384), F32)
    W6 = W6.at[:256, :256].set(Wsd2)
    W6 = W6.at[256:, 256:288].set(We2)

    def ep3(acc, ex):
        bb, w6 = ex
        t = jnp.maximum(acc + bb, 0.0)
        return (jnp.dot(t, w6, preferred_element_type=F32),)

    (B2,) = _adjmm(C_bf, B1, [(bias3, "full"), (W6, "full")], ep3, [384])

    bias4 = jnp.concatenate([bsd2, be2, jnp.zeros((96,), F32)]).reshape(1, -1)
    W7z = jnp.zeros((384, 128), F32).at[256:288, 0:64].set(Wg3)
    W7d = jnp.zeros((128, 128), F32).at[:, 64:72].set(weight1)

    def ep4(acc, ex):
        bb, w7z, w7d, dcb = ex
        z2 = acc + bb
        p1 = (jnp.dot(z2, w7z, preferred_element_type=F32)
              + jnp.dot(dcb, w7d, preferred_element_type=F32))
        return z2[:, :256], z2[:, 256:288], p1

    z_hat_emb, h_enc, P1 = _adjmm(C_bf, B2,
                                  [(bias4, "full"), (W7z, "full"),
                                   (W7d, "full"), (dec, "i")],
                                  ep4, [256, 32, 128])

    W8 = jnp.zeros((128, 256), F32)
    W8 = W8.at[0:64, 0:128].set(Wg4)
    W8 = W8.at[64:72, 128:256].set(weight2)

    def ep5(acc, ex):
        (w8,) = ex
        colid = lax.broadcasted_iota(jnp.int32, acc.shape, 1)
        q1 = jnp.where(colid < 64, jnp.maximum(acc, 0.0), acc)
        return q1, jnp.dot(q1, w8, preferred_element_type=F32)

    Q1, P2, adj_bf = _adjmm(adj_l, P1, [(W8, "full")], ep5, [128, 256],
                            emit_pbf=True)

    W9 = jnp.zeros((256, 256), F32).at[0:128, :].set(Wg5)

    def ep6(acc, ex):
        w9, he, eg, q1ex = ex
        colid = lax.broadcasted_iota(jnp.int32, acc.shape, 1)
        q2 = jnp.where(colid < 128, jnp.maximum(acc, 0.0), acc)
        zc = jnp.concatenate([he, eg, q1ex[:, 64:72]], axis=1)
        zp = jnp.concatenate([zc, jnp.zeros((zc.shape[0], 56), F32)], axis=1)
        return (jnp.dot(q2, w9, preferred_element_type=F32),
                acc[:, 128:], zc, zp)

    p3, dec_hat, z, z_pad = _adjmm(
        adj_bf, P2,
        [(W9, "full"), (h_enc, "i"), (emb32, "i"), (Q1, "i")],
        ep6, [256, 128, 72, 128])

    cl_pad = jnp.pad(cluster_layer, ((0, 6), (0, 56)))

    def ep7(acc, ex):
        zp, cl = ex
        z2s = jnp.sum(zp * zp, axis=1, keepdims=True)
        c2s = jnp.sum(cl * cl, axis=1)[None, :]
        cross = jnp.dot(zp, cl.T, preferred_element_type=F32)
        dist = z2s - 2.0 * cross + c2s
        qv = 1.0 / (1.0 + dist + 1e-8)
        qv = qv * qv / 2.0
        colid = lax.broadcasted_iota(jnp.int32, qv.shape, 1)
        qv = jnp.where(colid < 10, qv, 0.0)
        qn = qv / jnp.sum(qv, axis=1, keepdims=True)
        return acc, qn

    z_hat, qfull = _adjmm(adj_bf, p3, [(z_pad, "i"), (cl_pad, "full")],
                          ep7, [256, 16])

    return (z_hat, qfull[:, :10], z, z_hat_emb, dec_hat)


# speedup vs baseline: 26.5772x; 1.0431x over previous
import functools

import jax
import jax.numpy as jnp
from jax import lax
from jax.experimental import pallas as pl
from jax.experimental.pallas import tpu as pltpu
from jax.experimental.pallas import tpu_sc as plsc

N = 4096
E = 131072
THR = 0.1
BM = 1024
NB = N // BM
F32 = jnp.float32


NTILES = 16
EPT = E // NTILES
NCHUNKS = 16
CROWS = N // NCHUNKS
CWORDS = CROWS * N
DUMPW = 2048
ZN = CWORDS // NTILES
ZROWS = CROWS // NTILES
ZB = 8192
IDX_ROWS = EPT // 128
BF16 = jnp.bfloat16


def _count_body(edges_hbm, out_hbm, row_v, col_v, idx_a, idx_b, ones_v, zer_v,
                buf_sh, sem):
    cid = lax.axis_index("c")
    sid = lax.axis_index("s")

    ebase = pl.multiple_of(sid * EPT, 8)
    cp_r = pltpu.async_copy(edges_hbm.at[0, pl.ds(ebase, EPT)], row_v, sem)
    cp_c = pltpu.async_copy(edges_hbm.at[1, pl.ds(ebase, EPT)], col_v, sem)

    for j in range(8):
        ones_v[pl.ds(j * 16, 16)] = jnp.ones((16,), F32)

    def zbody(i, _):
        zer_v[pl.ds(i * 16, 16)] = jnp.zeros((16,), F32)
        return 0

    lax.fori_loop(0, ZB // 16, zbody, 0)
    cp_r.wait()
    cp_c.wait()

    lane = lax.iota(jnp.int32, 16)

    def compute_idx(idx_v, base):
        def ibody(i, _):
            for j in range(8):
                e0 = i * 128 + j * 16
                r = row_v[pl.ds(e0, 16)]
                c = col_v[pl.ds(e0, 16)]
                full = r * N + c - base
                valid = (full >= 0) & (full < CWORDS)
                dump = CWORDS + ((e0 + lane) & (DUMPW - 1))
                idx_v[i, pl.ds(j * 16, 16)] = jnp.where(valid, full, dump)
            return 0

        lax.fori_loop(0, IDX_ROWS, ibody, 0)

    def chunk_base(ch):
        return pl.multiple_of((cid * (NCHUNKS // 2) + ch) * CWORDS, 8)

    bufs = [idx_a, idx_b]
    compute_idx(bufs[0], chunk_base(0))
    out_cps = []

    for ch in range(NCHUNKS // 2):
        base = chunk_base(ch)
        for cp in out_cps:
            cp.wait()
        out_cps = []

        zcps = [
            pltpu.async_copy(
                zer_v,
                buf_sh.at[pl.ds(pl.multiple_of(sid * ZN + zz * ZB, 8), ZB)],
                sem)
            for zz in range(ZN // ZB)]

        @pl.when(sid == 0)
        def _zd():
            pltpu.sync_copy(zer_v.at[pl.ds(0, DUMPW)],
                            buf_sh.at[pl.ds(CWORDS, DUMPW)])

        for cp in zcps:
            cp.wait()

        plsc.subcore_barrier()

        idx_v = bufs[ch % 2]
        scps = [pltpu.async_copy(ones_v, buf_sh.at[idx_v.at[jj]], sem,
                                 add=True)
                for jj in range(IDX_ROWS)]
        if ch + 1 < NCHUNKS // 2:
            compute_idx(bufs[(ch + 1) % 2], chunk_base(ch + 1))
        for cp in scps:
            cp.wait()

        plsc.subcore_barrier()

        dst = pl.multiple_of(base + sid * ZN, 8)
        out_cps.append(pltpu.async_copy(
            buf_sh.at[pl.ds(pl.multiple_of(sid * ZN, 8), ZN)],
            out_hbm.at[pl.ds(dst, ZN)], sem))

    for cp in out_cps:
        cp.wait()


def _build_count(edge_index):
    mesh = plsc.VectorSubcoreMesh(core_axis_name="c", subcore_axis_name="s")
    fn = pl.kernel(
        _count_body,
        out_type=jax.ShapeDtypeStruct((N * N,), F32),
        mesh=mesh,
        scratch_types=[
            pltpu.VMEM((EPT,), jnp.int32),
            pltpu.VMEM((EPT,), jnp.int32),
            pltpu.VMEM((IDX_ROWS, 128), jnp.int32),
            pltpu.VMEM((IDX_ROWS, 128), jnp.int32),
            pltpu.VMEM((128,), F32),
            pltpu.VMEM((ZB,), F32),
            pltpu.VMEM_SHARED((CWORDS + DUMPW,), F32),
            pltpu.SemaphoreType.DMA,
        ],
    )
    return fn(edge_index)


def _retile_body(x_ref, o_ref):
    o_ref[...] = x_ref[...].reshape(o_ref.shape).astype(BF16)


def _retile(cflat):
    rb = 512
    return pl.pallas_call(
        _retile_body,
        grid=(N // rb,),
        in_specs=[pl.BlockSpec((rb * N,), lambda i: (i,))],
        out_specs=pl.BlockSpec((rb, N), lambda i: (i, 0)),
        out_shape=jax.ShapeDtypeStruct((N, N), BF16),
    )(cflat)


def _cast_body(x_ref, o_ref):
    o_ref[...] = x_ref[...].astype(BF16)


def _cast_bf(x):
    rb = 512
    return pl.pallas_call(
        _cast_body,
        grid=(N // rb,),
        in_specs=[pl.BlockSpec((rb, N), lambda i: (i, 0))],
        out_specs=pl.BlockSpec((rb, N), lambda i: (i, 0)),
        out_shape=jax.ShapeDtypeStruct((N, N), BF16),
    )(x)


def _mm_body(x_ref, w_ref, h_ref, u_ref, *, hsplit):
    hh = jnp.dot(x_ref[...], w_ref[...], preferred_element_type=F32)
    h_ref[...] = hh[:, :hsplit]
    u_ref[...] = hh[:, hsplit:]


def _smallmm(x, w, hsplit):
    wtot = w.shape[1]
    return pl.pallas_call(
        functools.partial(_mm_body, hsplit=hsplit),
        grid=(N // BM,),
        in_specs=[pl.BlockSpec((BM, x.shape[1]), lambda i: (i, 0)),
                  pl.BlockSpec(w.shape, lambda i: (0, 0))],
        out_specs=[pl.BlockSpec((BM, hsplit), lambda i: (i, 0)),
                   pl.BlockSpec((BM, wtot - hsplit), lambda i: (i, 0))],
        out_shape=[jax.ShapeDtypeStruct((N, hsplit), F32),
                   jax.ShapeDtypeStruct((N, wtot - hsplit), F32)],
    )(x, w)


def _rownorm_body(x_ref, o_ref):
    x = x_ref[...]
    nrm = jnp.maximum(jnp.sqrt(jnp.sum(x * x, axis=1, keepdims=True)), 1e-8)
    o_ref[...] = x / nrm


def _rownorm(x):
    n, d = x.shape
    return pl.pallas_call(
        _rownorm_body,
        grid=(n // BM,),
        in_specs=[pl.BlockSpec((BM, d), lambda i: (i, 0))],
        out_specs=pl.BlockSpec((BM, d), lambda i: (i, 0)),
        out_shape=jax.ShapeDtypeStruct((n, d), F32),
    )(x)


def _simadj_body(*refs, with_prev):
    idx = 0
    xi = refs[idx]; idx += 1
    xj = refs[idx]; idx += 1
    cref = refs[idx]; idx += 1
    if with_prev:
        a1_ref = refs[idx]; idx += 1
    a_ref = refs[idx]; idx += 1
    dinv_ref = refs[idx]; idx += 1
    acc_ref = refs[idx]

    i = pl.program_id(0)
    j = pl.program_id(1)
    s = jnp.dot(xi[...], xj[...].T, preferred_element_type=F32)
    ri = lax.broadcasted_iota(jnp.int32, (BM, BM), 0) + (i - j) * BM
    ci = lax.broadcasted_iota(jnp.int32, (BM, BM), 1)
    m = (s >= THR) & (ri != ci)
    if with_prev:
        m = m & (a1_ref[...] > 0)
    c = cref[...]
    if c.dtype != F32:
        c = c.astype(F32)
    a = jnp.where(m, s, 0.0) * c
    a_ref[...] = a.astype(jnp.bfloat16)

    @pl.when(j == 0)
    def _z():
        acc_ref[...] = jnp.zeros_like(acc_ref)

    acc_ref[...] += jnp.sum(a, axis=1, keepdims=True)

    @pl.when(j == NB - 1)
    def _f():
        dinv_ref[...] = lax.rsqrt(acc_ref[...] + 1.0)


def _simadj(xn, c, a1=None):
    with_prev = a1 is not None
    args = [xn, xn, c]
    in_specs = [pl.BlockSpec((BM, xn.shape[1]), lambda i, j: (i, 0)),
                pl.BlockSpec((BM, xn.shape[1]), lambda i, j: (j, 0)),
                pl.BlockSpec((BM, BM), lambda i, j: (i, j))]
    if with_prev:
        args.append(a1)
        in_specs.append(pl.BlockSpec((BM, BM), lambda i, j: (i, j)))
    out_shapes = [jax.ShapeDtypeStruct((N, N), jnp.bfloat16),
                  jax.ShapeDtypeStruct((N, 1), F32)]
    out_specs = [pl.BlockSpec((BM, BM), lambda i, j: (i, j)),
                 pl.BlockSpec((BM, 1), lambda i, j: (i, 0))]
    return pl.pallas_call(
        functools.partial(_simadj_body, with_prev=with_prev),
        grid=(NB, NB),
        in_specs=in_specs,
        out_specs=out_specs,
        out_shape=out_shapes,
        scratch_shapes=[pltpu.VMEM((BM, 1), F32)],
    )(*args)


def _adjmm_body(*refs, n_b, n_extra, n_out, epilogue, bscale):
    idx = 0
    p_ref = refs[idx]; idx += 1
    b_refs = refs[idx:idx + n_b]; idx += n_b
    if bscale:
        bs_ref = refs[idx]; idx += 1
    extra_refs = refs[idx:idx + n_extra]; idx += n_extra
    out_refs = refs[idx:idx + n_out]; idx += n_out
    acc_ref = refs[idx]

    k = pl.program_id(1)

    @pl.when(k == 0)
    def _z():
        acc_ref[...] = jnp.zeros_like(acc_ref)

    if n_b == 1:
        bb = b_refs[0][...]
    else:
        bb = jnp.concatenate([br[...] for br in b_refs], axis=1)
    if bscale:
        bb = bs_ref[...] * bb
    acc_ref[...] += jnp.dot(p_ref[...], bb.astype(jnp.bfloat16),
                            preferred_element_type=F32)

    @pl.when(k == NB - 1)
    def _f():
        outs = epilogue(acc_ref[...], [er[...] for er in extra_refs])
        for o_ref, o in zip(out_refs, outs):
            o_ref[...] = o


def _adjmm(p, b, extras, epilogue, out_widths, bscale=None):
    bs = b if isinstance(b, (list, tuple)) else [b]
    w = sum(x.shape[1] for x in bs)
    args = [p] + list(bs)
    in_specs = [pl.BlockSpec((BM, BM), lambda i, k: (i, k))]
    for x in bs:
        in_specs.append(pl.BlockSpec((BM, x.shape[1]), lambda i, k: (k, 0)))
    if bscale is not None:
        args.append(bscale)
        in_specs.append(pl.BlockSpec((BM, 1), lambda i, k: (k, 0)))
    for arr, mode in extras:
        args.append(arr)
        if mode == "i":
            in_specs.append(pl.BlockSpec((BM, arr.shape[1]),
                                         lambda i, k: (i, 0)))
        else:
            in_specs.append(pl.BlockSpec(arr.shape, lambda i, k: (0, 0)))
    out_shapes = [jax.ShapeDtypeStruct((N, ow), F32) for ow in out_widths]
    out_specs = [pl.BlockSpec((BM, ow), lambda i, k: (i, 0))
                 for ow in out_widths]
    res = pl.pallas_call(
        functools.partial(_adjmm_body, n_b=len(bs), n_extra=len(extras),
                          n_out=len(out_widths), epilogue=epilogue,
                          bscale=bscale is not None),
        grid=(NB, NB),
        in_specs=in_specs,
        out_specs=out_specs,
        out_shape=out_shapes,
        scratch_shapes=[pltpu.VMEM((BM, w), F32)],
    )(*args)
    return res


def kernel(x, adj_l, dec, edge_index, Wse1, bse1, Wse2, bse2, Wsd1, bsd1,
           Wsd2, bsd2, We1, be1, We2, be2, Wg3, Wg4, Wg5, weight1, weight2,
           cluster_layer):
    cflat = _build_count(edge_index)
    adj_bf = _cast_bf(adj_l)
    xn = _rownorm(x)
    wcat = jnp.concatenate([Wse1, We1], axis=1)
    h, u1 = _smallmm(x, wcat, 256)
    C_bf = _retile(cflat)

    A1, dinv1 = _simadj(xn, C_bf)

    Wse2p = jnp.pad(Wse2, ((0, 0), (0, 96)))
    bse1r = bse1.reshape(1, -1)

    def ep1(acc, ex):
        dinv, hh, bb, w2p = ex
        h1 = jnp.maximum(dinv * acc + dinv * dinv * hh + bb, 0.0)
        nrm = jnp.maximum(jnp.sqrt(jnp.sum(h1 * h1, axis=1, keepdims=True)),
                          1e-8)
        return h1 / nrm, jnp.dot(h1, w2p, preferred_element_type=F32)

    h1n, h2p = _adjmm(A1, h,
                      [(dinv1, "i"), (h, "i"), (bse1r, "full"),
                       (Wse2p, "full")],
                      ep1, [256, 128], bscale=dinv1)

    A2, dinv2 = _simadj(h1n, C_bf, a1=A1)
    bse2p = jnp.pad(bse2, (0, 96)).reshape(1, -1)
    Wsd1p = jnp.pad(Wsd1, ((0, 96), (0, 0)))

    def ep2(acc, ex):
        dinv, hh, bb, wsd1p = ex
        e = dinv * acc + dinv * dinv * hh + bb
        return e[:, :32], jnp.dot(e, wsd1p, preferred_element_type=F32)

    emb32, g1 = _adjmm(A2, h2p,
                       [(dinv2, "i"), (h2p, "i"), (bse2p, "full"),
                        (Wsd1p, "full")],
                       ep2, [32, 256], bscale=dinv2)

    bias3 = jnp.concatenate([bsd1, be1]).reshape(1, -1)
    W6 = jnp.zeros((384, 384), F32)
    W6 = W6.at[:256, :256].set(Wsd2)
    W6 = W6.at[256:, 256:288].set(We2)

    def ep3(acc, ex):
        bb, w6 = ex
        t = jnp.maximum(acc + bb, 0.0)
        return (jnp.dot(t, w6, preferred_element_type=F32),)

    (B2,) = _adjmm(C_bf, [g1, u1], [(bias3, "full"), (W6, "full")],
                   ep3, [384])

    bias4 = jnp.concatenate([bsd2, be2, jnp.zeros((96,), F32)]).reshape(1, -1)
    W7z = jnp.zeros((384, 128), F32).at[256:288, 0:64].set(Wg3)
    W7d = jnp.zeros((128, 128), F32).at[:, 64:72].set(weight1)

    def ep4(acc, ex):
        bb, w7z, w7d, dcb = ex
        z2 = acc + bb
        p1 = (jnp.dot(z2, w7z, preferred_element_type=F32)
              + jnp.dot(dcb, w7d, preferred_element_type=F32))
        return z2[:, :256], z2[:, 256:288], p1

    z_hat_emb, h_enc, P1 = _adjmm(C_bf, B2,
                                  [(bias4, "full"), (W7z, "full"),
                                   (W7d, "full"), (dec, "i")],
                                  ep4, [256, 32, 128])

    W8 = jnp.zeros((128, 256), F32)
    W8 = W8.at[0:64, 0:128].set(Wg4)
    W8 = W8.at[64:72, 128:256].set(weight2)

    def ep5(acc, ex):
        (w8,) = ex
        colid = lax.broadcasted_iota(jnp.int32, acc.shape, 1)
        q1 = jnp.where(colid < 64, jnp.maximum(acc, 0.0), acc)
        return q1, jnp.dot(q1, w8, preferred_element_type=F32)

    Q1, P2 = _adjmm(adj_bf, P1, [(W8, "full")], ep5, [128, 256])

    W9 = jnp.zeros((256, 256), F32).at[0:128, :].set(Wg5)

    def ep6(acc, ex):
        w9, he, eg, q1ex = ex
        colid = lax.broadcasted_iota(jnp.int32, acc.shape, 1)
        q2 = jnp.where(colid < 128, jnp.maximum(acc, 0.0), acc)
        zc = jnp.concatenate([he, eg, q1ex[:, 64:72]], axis=1)
        return (jnp.dot(q2, w9, preferred_element_type=F32),
                acc[:, 128:], zc)

    p3, dec_hat, z = _adjmm(
        adj_bf, P2,
        [(W9, "full"), (h_enc, "i"), (emb32, "i"), (Q1, "i")],
        ep6, [256, 128, 72])

    cl_pad = jnp.pad(cluster_layer, ((0, 6), (0, 56)))

    def ep7(acc, ex):
        zc, cl = ex
        zp = jnp.concatenate([zc, jnp.zeros((zc.shape[0], 56), F32)], axis=1)
        z2s = jnp.sum(zp * zp, axis=1, keepdims=True)
        c2s = jnp.sum(cl * cl, axis=1)[None, :]
        cross = jnp.dot(zp, cl.T, preferred_element_type=F32)
        dist = z2s - 2.0 * cross + c2s
        qv = 1.0 / (1.0 + dist + 1e-8)
        qv = qv * qv / 2.0
        colid = lax.broadcasted_iota(jnp.int32, qv.shape, 1)
        qv = jnp.where(colid < 10, qv, 0.0)
        qn = qv / jnp.sum(qv, axis=1, keepdims=True)
        return acc, qn

    z_hat, qfull = _adjmm(adj_bf, p3, [(z, "i"), (cl_pad, "full")],
                          ep7, [256, 16])

    return (z_hat, qfull[:, :10], z, z_hat_emb, dec_hat)


# speedup vs baseline: 29.4821x; 1.1093x over previous
import functools

import jax
import jax.numpy as jnp
from jax import lax
from jax.experimental import pallas as pl
from jax.experimental.pallas import tpu as pltpu
from jax.experimental.pallas import tpu_sc as plsc

N = 4096
E = 131072
THR = 0.1
BM = 1024
NB = N // BM
F32 = jnp.float32


NTILES = 16
EPT = E // NTILES
NCHUNKS = 16
CROWS = N // NCHUNKS
CWORDS = CROWS * N
DUMPW = 8192
ZN = CWORDS // NTILES
ZROWS = CROWS // NTILES
ZB = 8192
IDX_ROWS = EPT // 128
BF16 = jnp.bfloat16


def _count_body(edges_hbm, out_hbm, row_v, col_v, idx_a, idx_b, ones_v, zer_v,
                buf_sh, sem):
    cid = lax.axis_index("c")
    sid = lax.axis_index("s")

    ebase = pl.multiple_of(sid * EPT, 8)
    cp_r = pltpu.async_copy(edges_hbm.at[0, pl.ds(ebase, EPT)], row_v, sem)
    cp_c = pltpu.async_copy(edges_hbm.at[1, pl.ds(ebase, EPT)], col_v, sem)

    for j in range(8):
        ones_v[pl.ds(j * 16, 16)] = jnp.ones((16,), F32)

    def zbody(i, _):
        zer_v[pl.ds(i * 16, 16)] = jnp.zeros((16,), F32)
        return 0

    lax.fori_loop(0, ZB // 16, zbody, 0)
    cp_r.wait()
    cp_c.wait()

    lane = lax.iota(jnp.int32, 16)

    def compute_idx(idx_v, base):
        def ibody(i, _):
            for j in range(8):
                e0 = i * 128 + j * 16
                r = row_v[pl.ds(e0, 16)]
                c = col_v[pl.ds(e0, 16)]
                full = r * N + c - base
                valid = (full >= 0) & (full < CWORDS)
                dump = CWORDS + ((e0 + lane) & (DUMPW - 1))
                idx_v[i, pl.ds(j * 16, 16)] = jnp.where(valid, full, dump)
            return 0

        lax.fori_loop(0, IDX_ROWS, ibody, 0)

    def chunk_base(ch):
        return pl.multiple_of((cid * (NCHUNKS // 2) + ch) * CWORDS, 8)

    bufs = [idx_a, idx_b]
    compute_idx(bufs[0], chunk_base(0))
    out_cps = []

    for ch in range(NCHUNKS // 2):
        base = chunk_base(ch)
        for cp in out_cps:
            cp.wait()
        out_cps = []

        zcps = [
            pltpu.async_copy(
                zer_v,
                buf_sh.at[pl.ds(pl.multiple_of(sid * ZN + zz * ZB, 8), ZB)],
                sem)
            for zz in range(ZN // ZB)]

        @pl.when(sid == 0)
        def _zd():
            pltpu.sync_copy(zer_v, buf_sh.at[pl.ds(CWORDS, DUMPW)])

        for cp in zcps:
            cp.wait()

        plsc.subcore_barrier()

        idx_v = bufs[ch % 2]
        scps = [pltpu.async_copy(ones_v, buf_sh.at[idx_v.at[jj]], sem,
                                 add=True)
                for jj in range(IDX_ROWS)]
        if ch + 1 < NCHUNKS // 2:
            compute_idx(bufs[(ch + 1) % 2], chunk_base(ch + 1))
        for cp in scps:
            cp.wait()

        plsc.subcore_barrier()

        dst = pl.multiple_of(base + sid * ZN, 8)
        out_cps.append(pltpu.async_copy(
            buf_sh.at[pl.ds(pl.multiple_of(sid * ZN, 8), ZN)],
            out_hbm.at[pl.ds(dst, ZN)], sem))

    for cp in out_cps:
        cp.wait()


def _build_count(edge_index):
    mesh = plsc.VectorSubcoreMesh(core_axis_name="c", subcore_axis_name="s")
    fn = pl.kernel(
        _count_body,
        out_type=jax.ShapeDtypeStruct((N * N,), F32),
        mesh=mesh,
        scratch_types=[
            pltpu.VMEM((EPT,), jnp.int32),
            pltpu.VMEM((EPT,), jnp.int32),
            pltpu.VMEM((IDX_ROWS, 128), jnp.int32),
            pltpu.VMEM((IDX_ROWS, 128), jnp.int32),
            pltpu.VMEM((128,), F32),
            pltpu.VMEM((ZB,), F32),
            pltpu.VMEM_SHARED((CWORDS + DUMPW,), F32),
            pltpu.SemaphoreType.DMA,
        ],
    )
    return fn(edge_index)


def _retile_body(x_ref, o_ref):
    o_ref[...] = x_ref[...].reshape(o_ref.shape).astype(BF16)


def _retile(cflat):
    rb = 512
    return pl.pallas_call(
        _retile_body,
        grid=(N // rb,),
        in_specs=[pl.BlockSpec((rb * N,), lambda i: (i,))],
        out_specs=pl.BlockSpec((rb, N), lambda i: (i, 0)),
        out_shape=jax.ShapeDtypeStruct((N, N), BF16),
    )(cflat)


def _cast_body(x_ref, o_ref):
    o_ref[...] = x_ref[...].astype(BF16)


def _cast_bf(x):
    rb = 512
    return pl.pallas_call(
        _cast_body,
        grid=(N // rb,),
        in_specs=[pl.BlockSpec((rb, N), lambda i: (i, 0))],
        out_specs=pl.BlockSpec((rb, N), lambda i: (i, 0)),
        out_shape=jax.ShapeDtypeStruct((N, N), BF16),
    )(x)


def _mm_body(x_ref, w_ref, h_ref, u_ref, *, hsplit):
    hh = jnp.dot(x_ref[...], w_ref[...], preferred_element_type=F32)
    h_ref[...] = hh[:, :hsplit]
    u_ref[...] = hh[:, hsplit:]


def _smallmm(x, w, hsplit):
    wtot = w.shape[1]
    return pl.pallas_call(
        functools.partial(_mm_body, hsplit=hsplit),
        grid=(N // BM,),
        in_specs=[pl.BlockSpec((BM, x.shape[1]), lambda i: (i, 0)),
                  pl.BlockSpec(w.shape, lambda i: (0, 0))],
        out_specs=[pl.BlockSpec((BM, hsplit), lambda i: (i, 0)),
                   pl.BlockSpec((BM, wtot - hsplit), lambda i: (i, 0))],
        out_shape=[jax.ShapeDtypeStruct((N, hsplit), F32),
                   jax.ShapeDtypeStruct((N, wtot - hsplit), F32)],
    )(x, w)


def _rownorm_body(x_ref, o_ref):
    x = x_ref[...]
    nrm = jnp.maximum(jnp.sqrt(jnp.sum(x * x, axis=1, keepdims=True)), 1e-8)
    o_ref[...] = x / nrm


def _rownorm(x):
    n, d = x.shape
    return pl.pallas_call(
        _rownorm_body,
        grid=(n // BM,),
        in_specs=[pl.BlockSpec((BM, d), lambda i: (i, 0))],
        out_specs=pl.BlockSpec((BM, d), lambda i: (i, 0)),
        out_shape=jax.ShapeDtypeStruct((n, d), F32),
    )(x)


def _simadj_body(*refs, with_prev):
    idx = 0
    xi = refs[idx]; idx += 1
    xj = refs[idx]; idx += 1
    cref = refs[idx]; idx += 1
    if with_prev:
        a1_ref = refs[idx]; idx += 1
    a_ref = refs[idx]; idx += 1
    dinv_ref = refs[idx]; idx += 1
    acc_ref = refs[idx]

    i = pl.program_id(0)
    j = pl.program_id(1)
    s = jnp.dot(xi[...], xj[...].T, preferred_element_type=F32)
    ri = lax.broadcasted_iota(jnp.int32, (BM, BM), 0) + (i - j) * BM
    ci = lax.broadcasted_iota(jnp.int32, (BM, BM), 1)
    m = (s >= THR) & (ri != ci)
    if with_prev:
        m = m & (a1_ref[...] > 0)
    c = cref[...]
    if c.dtype != F32:
        c = c.astype(F32)
    a = jnp.where(m, s, 0.0) * c
    a_ref[...] = a.astype(jnp.bfloat16)

    @pl.when(j == 0)
    def _z():
        acc_ref[...] = jnp.zeros_like(acc_ref)

    acc_ref[...] += jnp.sum(a, axis=1, keepdims=True)

    @pl.when(j == NB - 1)
    def _f():
        dinv_ref[...] = lax.rsqrt(acc_ref[...] + 1.0)


def _simadj(xn, c, a1=None):
    with_prev = a1 is not None
    args = [xn, xn, c]
    in_specs = [pl.BlockSpec((BM, xn.shape[1]), lambda i, j: (i, 0)),
                pl.BlockSpec((BM, xn.shape[1]), lambda i, j: (j, 0)),
                pl.BlockSpec((BM, BM), lambda i, j: (i, j))]
    if with_prev:
        args.append(a1)
        in_specs.append(pl.BlockSpec((BM, BM), lambda i, j: (i, j)))
    out_shapes = [jax.ShapeDtypeStruct((N, N), jnp.bfloat16),
                  jax.ShapeDtypeStruct((N, 1), F32)]
    out_specs = [pl.BlockSpec((BM, BM), lambda i, j: (i, j)),
                 pl.BlockSpec((BM, 1), lambda i, j: (i, 0))]
    return pl.pallas_call(
        functools.partial(_simadj_body, with_prev=with_prev),
        grid=(NB, NB),
        in_specs=in_specs,
        out_specs=out_specs,
        out_shape=out_shapes,
        scratch_shapes=[pltpu.VMEM((BM, 1), F32)],
    )(*args)


def _adjmm_body(*refs, n_b, n_extra, n_out, epilogue, bscale):
    idx = 0
    p_ref = refs[idx]; idx += 1
    b_refs = refs[idx:idx + n_b]; idx += n_b
    if bscale:
        bs_ref = refs[idx]; idx += 1
    extra_refs = refs[idx:idx + n_extra]; idx += n_extra
    out_refs = refs[idx:idx + n_out]; idx += n_out
    acc_ref = refs[idx]

    k = pl.program_id(1)

    @pl.when(k == 0)
    def _z():
        acc_ref[...] = jnp.zeros_like(acc_ref)

    if n_b == 1:
        bb = b_refs[0][...]
    else:
        bb = jnp.concatenate([br[...] for br in b_refs], axis=1)
    if bscale:
        bb = bs_ref[...] * bb
    acc_ref[...] += jnp.dot(p_ref[...], bb.astype(jnp.bfloat16),
                            preferred_element_type=F32)

    @pl.when(k == pl.num_programs(1) - 1)
    def _f():
        outs = epilogue(acc_ref[...], [er[...] for er in extra_refs])
        for o_ref, o in zip(out_refs, outs):
            o_ref[...] = o


def _adjmm(p, b, extras, epilogue, out_widths, bscale=None):
    bm = 2048
    bs = b if isinstance(b, (list, tuple)) else [b]
    w = sum(x.shape[1] for x in bs)
    args = [p] + list(bs)
    in_specs = [pl.BlockSpec((bm, bm), lambda i, k: (i, k))]
    for x in bs:
        in_specs.append(pl.BlockSpec((bm, x.shape[1]), lambda i, k: (k, 0)))
    if bscale is not None:
        args.append(bscale)
        in_specs.append(pl.BlockSpec((bm, 1), lambda i, k: (k, 0)))
    for arr, mode in extras:
        args.append(arr)
        if mode == "i":
            in_specs.append(pl.BlockSpec((bm, arr.shape[1]),
                                         lambda i, k: (i, 0)))
        else:
            in_specs.append(pl.BlockSpec(arr.shape, lambda i, k: (0, 0)))
    out_shapes = [jax.ShapeDtypeStruct((N, ow), F32) for ow in out_widths]
    out_specs = [pl.BlockSpec((bm, ow), lambda i, k: (i, 0))
                 for ow in out_widths]
    res = pl.pallas_call(
        functools.partial(_adjmm_body, n_b=len(bs), n_extra=len(extras),
                          n_out=len(out_widths), epilogue=epilogue,
                          bscale=bscale is not None),
        grid=(N // bm, N // bm),
        in_specs=in_specs,
        out_specs=out_specs,
        out_shape=out_shapes,
        scratch_shapes=[pltpu.VMEM((bm, w), F32)],
    )(*args)
    return res


def kernel(x, adj_l, dec, edge_index, Wse1, bse1, Wse2, bse2, Wsd1, bsd1,
           Wsd2, bsd2, We1, be1, We2, be2, Wg3, Wg4, Wg5, weight1, weight2,
           cluster_layer):
    cflat = _build_count(edge_index)
    adj_bf = _cast_bf(adj_l)
    xn = _rownorm(x)
    wcat = jnp.concatenate([Wse1, We1], axis=1)
    h, u1 = _smallmm(x, wcat, 256)
    C_bf = _retile(cflat)

    A1, dinv1 = _simadj(xn, C_bf)

    Wse2p = jnp.pad(Wse2, ((0, 0), (0, 96)))
    bse1r = bse1.reshape(1, -1)

    def ep1(acc, ex):
        dinv, hh, bb, w2p = ex
        h1 = jnp.maximum(dinv * acc + dinv * dinv * hh + bb, 0.0)
        nrm = jnp.maximum(jnp.sqrt(jnp.sum(h1 * h1, axis=1, keepdims=True)),
                          1e-8)
        return h1 / nrm, jnp.dot(h1, w2p, preferred_element_type=F32)

    h1n, h2p = _adjmm(A1, h,
                      [(dinv1, "i"), (h, "i"), (bse1r, "full"),
                       (Wse2p, "full")],
                      ep1, [256, 128], bscale=dinv1)

    A2, dinv2 = _simadj(h1n, C_bf, a1=A1)
    bse2p = jnp.pad(bse2, (0, 96)).reshape(1, -1)
    Wsd1p = jnp.pad(Wsd1, ((0, 96), (0, 0)))

    def ep2(acc, ex):
        dinv, hh, bb, wsd1p = ex
        e = dinv * acc + dinv * dinv * hh + bb
        return e[:, :32], jnp.dot(e, wsd1p, preferred_element_type=F32)

    emb32, g1 = _adjmm(A2, h2p,
                       [(dinv2, "i"), (h2p, "i"), (bse2p, "full"),
                        (Wsd1p, "full")],
                       ep2, [32, 256], bscale=dinv2)

    bias3 = jnp.concatenate([bsd1, be1]).reshape(1, -1)
    W6 = jnp.zeros((384, 384), F32)
    W6 = W6.at[:256, :256].set(Wsd2)
    W6 = W6.at[256:, 256:288].set(We2)

    def ep3(acc, ex):
        bb, w6 = ex
        t = jnp.maximum(acc + bb, 0.0)
        return (jnp.dot(t, w6, preferred_element_type=F32),)

    (B2,) = _adjmm(C_bf, [g1, u1], [(bias3, "full"), (W6, "full")],
                   ep3, [384])

    bias4 = jnp.concatenate([bsd2, be2, jnp.zeros((96,), F32)]).reshape(1, -1)
    W7z = jnp.zeros((384, 128), F32).at[256:288, 0:64].set(Wg3)
    W7d = jnp.zeros((128, 128), F32).at[:, 64:72].set(weight1)

    def ep4(acc, ex):
        bb, w7z, w7d, dcb = ex
        z2 = acc + bb
        p1 = (jnp.dot(z2, w7z, preferred_element_type=F32)
              + jnp.dot(dcb, w7d, preferred_element_type=F32))
        return z2[:, :256], z2[:, 256:288], p1

    z_hat_emb, h_enc, P1 = _adjmm(C_bf, B2,
                                  [(bias4, "full"), (W7z, "full"),
                                   (W7d, "full"), (dec, "i")],
                                  ep4, [256, 32, 128])

    W8 = jnp.zeros((128, 256), F32)
    W8 = W8.at[0:64, 0:128].set(Wg4)
    W8 = W8.at[64:72, 128:256].set(weight2)

    def ep5(acc, ex):
        (w8,) = ex
        colid = lax.broadcasted_iota(jnp.int32, acc.shape, 1)
        q1 = jnp.where(colid < 64, jnp.maximum(acc, 0.0), acc)
        return q1, jnp.dot(q1, w8, preferred_element_type=F32)

    Q1, P2 = _adjmm(adj_bf, P1, [(W8, "full")], ep5, [128, 256])

    W9 = jnp.zeros((256, 256), F32).at[0:128, :].set(Wg5)

    def ep6(acc, ex):
        w9, he, eg, q1ex = ex
        colid = lax.broadcasted_iota(jnp.int32, acc.shape, 1)
        q2 = jnp.where(colid < 128, jnp.maximum(acc, 0.0), acc)
        zc = jnp.concatenate([he, eg, q1ex[:, 64:72]], axis=1)
        return (jnp.dot(q2, w9, preferred_element_type=F32),
                acc[:, 128:], zc)

    p3, dec_hat, z = _adjmm(
        adj_bf, P2,
        [(W9, "full"), (h_enc, "i"), (emb32, "i"), (Q1, "i")],
        ep6, [256, 128, 72])

    cl_pad = jnp.pad(cluster_layer, ((0, 6), (0, 56)))

    def ep7(acc, ex):
        zc, cl = ex
        zp = jnp.concatenate([zc, jnp.zeros((zc.shape[0], 56), F32)], axis=1)
        z2s = jnp.sum(zp * zp, axis=1, keepdims=True)
        c2s = jnp.sum(cl * cl, axis=1)[None, :]
        cross = jnp.dot(zp, cl.T, preferred_element_type=F32)
        dist = z2s - 2.0 * cross + c2s
        qv = 1.0 / (1.0 + dist + 1e-8)
        qv = qv * qv / 2.0
        colid = lax.broadcasted_iota(jnp.int32, qv.shape, 1)
        qv = jnp.where(colid < 10, qv, 0.0)
        qn = qv / jnp.sum(qv, axis=1, keepdims=True)
        return acc, qn

    z_hat, qfull = _adjmm(adj_bf, p3, [(z, "i"), (cl_pad, "full")],
                          ep7, [256, 16])

    return (z_hat, qfull[:, :10], z, z_hat_emb, dec_hat)


# speedup vs baseline: 30.7412x; 1.0427x over previous
import functools

import jax
import jax.numpy as jnp
from jax import lax
from jax.experimental import pallas as pl
from jax.experimental.pallas import tpu as pltpu
from jax.experimental.pallas import tpu_sc as plsc

N = 4096
E = 131072
THR = 0.1
BM = 1024
NB = N // BM
F32 = jnp.float32


NTILES = 16
EPT = E // NTILES
NCHUNKS = 16
CROWS = N // NCHUNKS
CWORDS = CROWS * N
DUMPW = 8192
ZN = CWORDS // NTILES
ZROWS = CROWS // NTILES
ZB = 8192
IDX_ROWS = EPT // 128
BF16 = jnp.bfloat16


def _count_body(edges_hbm, out_hbm, row_v, col_v, idx_a, idx_b, ones_v, zer_v,
                buf_sh, sem):
    cid = lax.axis_index("c")
    sid = lax.axis_index("s")

    ebase = pl.multiple_of(sid * EPT, 8)
    cp_r = pltpu.async_copy(edges_hbm.at[0, pl.ds(ebase, EPT)], row_v, sem)
    cp_c = pltpu.async_copy(edges_hbm.at[1, pl.ds(ebase, EPT)], col_v, sem)

    for j in range(8):
        ones_v[pl.ds(j * 16, 16)] = jnp.ones((16,), F32)

    def zbody(i, _):
        zer_v[pl.ds(i * 16, 16)] = jnp.zeros((16,), F32)
        return 0

    lax.fori_loop(0, ZB // 16, zbody, 0)
    cp_r.wait()
    cp_c.wait()

    lane = lax.iota(jnp.int32, 16)

    def compute_idx(idx_v, base):
        def ibody(i, _):
            for j in range(8):
                e0 = i * 128 + j * 16
                r = row_v[pl.ds(e0, 16)]
                c = col_v[pl.ds(e0, 16)]
                full = r * N + c - base
                valid = (full >= 0) & (full < CWORDS)
                dump = CWORDS + ((e0 + lane) & (DUMPW - 1))
                idx_v[i, pl.ds(j * 16, 16)] = jnp.where(valid, full, dump)
            return 0

        lax.fori_loop(0, IDX_ROWS, ibody, 0)

    def chunk_base(ch):
        return pl.multiple_of((cid * (NCHUNKS // 2) + ch) * CWORDS, 8)

    bufs = [idx_a, idx_b]
    compute_idx(bufs[0], chunk_base(0))
    out_cps = []

    for ch in range(NCHUNKS // 2):
        base = chunk_base(ch)
        for cp in out_cps:
            cp.wait()
        out_cps = []

        zcps = [
            pltpu.async_copy(
                zer_v,
                buf_sh.at[pl.ds(pl.multiple_of(sid * ZN + zz * ZB, 8), ZB)],
                sem)
            for zz in range(ZN // ZB)]

        @pl.when(sid == 0)
        def _zd():
            pltpu.sync_copy(zer_v, buf_sh.at[pl.ds(CWORDS, DUMPW)])

        for cp in zcps:
            cp.wait()

        plsc.subcore_barrier()

        idx_v = bufs[ch % 2]
        scps = [pltpu.async_copy(ones_v, buf_sh.at[idx_v.at[jj]], sem,
                                 add=True)
                for jj in range(IDX_ROWS)]
        if ch + 1 < NCHUNKS // 2:
            compute_idx(bufs[(ch + 1) % 2], chunk_base(ch + 1))
        for cp in scps:
            cp.wait()

        plsc.subcore_barrier()

        dst = pl.multiple_of(base + sid * ZN, 8)
        out_cps.append(pltpu.async_copy(
            buf_sh.at[pl.ds(pl.multiple_of(sid * ZN, 8), ZN)],
            out_hbm.at[pl.ds(dst, ZN)], sem))

    for cp in out_cps:
        cp.wait()


def _build_count(edge_index):
    mesh = plsc.VectorSubcoreMesh(core_axis_name="c", subcore_axis_name="s")
    fn = pl.kernel(
        _count_body,
        out_type=jax.ShapeDtypeStruct((N * N,), F32),
        mesh=mesh,
        scratch_types=[
            pltpu.VMEM((EPT,), jnp.int32),
            pltpu.VMEM((EPT,), jnp.int32),
            pltpu.VMEM((IDX_ROWS, 128), jnp.int32),
            pltpu.VMEM((IDX_ROWS, 128), jnp.int32),
            pltpu.VMEM((128,), F32),
            pltpu.VMEM((ZB,), F32),
            pltpu.VMEM_SHARED((CWORDS + DUMPW,), F32),
            pltpu.SemaphoreType.DMA,
        ],
    )
    return fn(edge_index)


def _retile_body(x_ref, w_ref, c_ref, a_ref, dinv_ref):
    c = x_ref[...].reshape(c_ref.shape)
    c_ref[...] = c.astype(BF16)
    a = w_ref[...].astype(F32) * c
    a_ref[...] = a.astype(BF16)
    dinv_ref[...] = lax.rsqrt(jnp.sum(a, axis=1, keepdims=True) + 1.0)


def _retile(cflat, w1d):
    rb = 512
    return pl.pallas_call(
        _retile_body,
        grid=(N // rb,),
        in_specs=[pl.BlockSpec((rb * N,), lambda i: (i,)),
                  pl.BlockSpec((rb, N), lambda i: (i, 0))],
        out_specs=[pl.BlockSpec((rb, N), lambda i: (i, 0)),
                   pl.BlockSpec((rb, N), lambda i: (i, 0)),
                   pl.BlockSpec((rb, 1), lambda i: (i, 0))],
        out_shape=[jax.ShapeDtypeStruct((N, N), BF16),
                   jax.ShapeDtypeStruct((N, N), BF16),
                   jax.ShapeDtypeStruct((N, 1), F32)],
    )(cflat, w1d)


def _simw_body(xi_ref, xj_ref, w_ref):
    i = pl.program_id(0)
    j = pl.program_id(1)
    s = jnp.dot(xi_ref[...], xj_ref[...].T, preferred_element_type=F32)
    ri = lax.broadcasted_iota(jnp.int32, s.shape, 0) + (i - j) * BM
    ci = lax.broadcasted_iota(jnp.int32, s.shape, 1)
    m = (s >= THR) & (ri != ci)
    w_ref[...] = jnp.where(m, s, 0.0).astype(BF16)


def _simw(xn):
    return pl.pallas_call(
        _simw_body,
        grid=(NB, NB),
        in_specs=[pl.BlockSpec((BM, xn.shape[1]), lambda i, j: (i, 0)),
                  pl.BlockSpec((BM, xn.shape[1]), lambda i, j: (j, 0))],
        out_specs=pl.BlockSpec((BM, BM), lambda i, j: (i, j)),
        out_shape=jax.ShapeDtypeStruct((N, N), BF16),
    )(xn, xn)


def _cast_body(x_ref, o_ref):
    o_ref[...] = x_ref[...].astype(BF16)


def _cast_bf(x):
    rb = 512
    return pl.pallas_call(
        _cast_body,
        grid=(N // rb,),
        in_specs=[pl.BlockSpec((rb, N), lambda i: (i, 0))],
        out_specs=pl.BlockSpec((rb, N), lambda i: (i, 0)),
        out_shape=jax.ShapeDtypeStruct((N, N), BF16),
    )(x)


def _mm_body(x_ref, w_ref, h_ref, u_ref, *, hsplit):
    hh = jnp.dot(x_ref[...], w_ref[...], preferred_element_type=F32)
    h_ref[...] = hh[:, :hsplit]
    u_ref[...] = hh[:, hsplit:]


def _smallmm(x, w, hsplit):
    wtot = w.shape[1]
    return pl.pallas_call(
        functools.partial(_mm_body, hsplit=hsplit),
        grid=(N // BM,),
        in_specs=[pl.BlockSpec((BM, x.shape[1]), lambda i: (i, 0)),
                  pl.BlockSpec(w.shape, lambda i: (0, 0))],
        out_specs=[pl.BlockSpec((BM, hsplit), lambda i: (i, 0)),
                   pl.BlockSpec((BM, wtot - hsplit), lambda i: (i, 0))],
        out_shape=[jax.ShapeDtypeStruct((N, hsplit), F32),
                   jax.ShapeDtypeStruct((N, wtot - hsplit), F32)],
    )(x, w)


def _rownorm_body(x_ref, o_ref):
    x = x_ref[...]
    nrm = jnp.maximum(jnp.sqrt(jnp.sum(x * x, axis=1, keepdims=True)), 1e-8)
    o_ref[...] = x / nrm


def _rownorm(x):
    n, d = x.shape
    return pl.pallas_call(
        _rownorm_body,
        grid=(n // BM,),
        in_specs=[pl.BlockSpec((BM, d), lambda i: (i, 0))],
        out_specs=pl.BlockSpec((BM, d), lambda i: (i, 0)),
        out_shape=jax.ShapeDtypeStruct((n, d), F32),
    )(x)


def _simadj_body(*refs, with_prev):
    idx = 0
    xi = refs[idx]; idx += 1
    xj = refs[idx]; idx += 1
    cref = refs[idx]; idx += 1
    if with_prev:
        a1_ref = refs[idx]; idx += 1
    a_ref = refs[idx]; idx += 1
    dinv_ref = refs[idx]; idx += 1
    acc_ref = refs[idx]

    i = pl.program_id(0)
    j = pl.program_id(1)
    s = jnp.dot(xi[...], xj[...].T, preferred_element_type=F32)
    ri = lax.broadcasted_iota(jnp.int32, (BM, BM), 0) + (i - j) * BM
    ci = lax.broadcasted_iota(jnp.int32, (BM, BM), 1)
    m = (s >= THR) & (ri != ci)
    if with_prev:
        m = m & (a1_ref[...] > 0)
    c = cref[...]
    if c.dtype != F32:
        c = c.astype(F32)
    a = jnp.where(m, s, 0.0) * c
    a_ref[...] = a.astype(jnp.bfloat16)

    @pl.when(j == 0)
    def _z():
        acc_ref[...] = jnp.zeros_like(acc_ref)

    acc_ref[...] += jnp.sum(a, axis=1, keepdims=True)

    @pl.when(j == NB - 1)
    def _f():
        dinv_ref[...] = lax.rsqrt(acc_ref[...] + 1.0)


def _simadj(xn, c, a1=None):
    with_prev = a1 is not None
    args = [xn, xn, c]
    in_specs = [pl.BlockSpec((BM, xn.shape[1]), lambda i, j: (i, 0)),
                pl.BlockSpec((BM, xn.shape[1]), lambda i, j: (j, 0)),
                pl.BlockSpec((BM, BM), lambda i, j: (i, j))]
    if with_prev:
        args.append(a1)
        in_specs.append(pl.BlockSpec((BM, BM), lambda i, j: (i, j)))
    out_shapes = [jax.ShapeDtypeStruct((N, N), jnp.bfloat16),
                  jax.ShapeDtypeStruct((N, 1), F32)]
    out_specs = [pl.BlockSpec((BM, BM), lambda i, j: (i, j)),
                 pl.BlockSpec((BM, 1), lambda i, j: (i, 0))]
    return pl.pallas_call(
        functools.partial(_simadj_body, with_prev=with_prev),
        grid=(NB, NB),
        in_specs=in_specs,
        out_specs=out_specs,
        out_shape=out_shapes,
        scratch_shapes=[pltpu.VMEM((BM, 1), F32)],
    )(*args)


def _adjmm_body(*refs, n_b, n_extra, n_out, epilogue, bscale):
    idx = 0
    p_ref = refs[idx]; idx += 1
    b_refs = refs[idx:idx + n_b]; idx += n_b
    if bscale:
        bs_ref = refs[idx]; idx += 1
    extra_refs = refs[idx:idx + n_extra]; idx += n_extra
    out_refs = refs[idx:idx + n_out]; idx += n_out
    acc_ref = refs[idx]

    k = pl.program_id(1)

    @pl.when(k == 0)
    def _z():
        acc_ref[...] = jnp.zeros_like(acc_ref)

    if n_b == 1:
        bb = b_refs[0][...]
    else:
        bb = jnp.concatenate([br[...] for br in b_refs], axis=1)
    if bscale:
        bb = bs_ref[...] * bb
    acc_ref[...] += jnp.dot(p_ref[...], bb.astype(jnp.bfloat16),
                            preferred_element_type=F32)

    @pl.when(k == pl.num_programs(1) - 1)
    def _f():
        outs = epilogue(acc_ref[...], [er[...] for er in extra_refs])
        for o_ref, o in zip(out_refs, outs):
            o_ref[...] = o


def _adjmm(p, b, extras, epilogue, out_widths, bscale=None):
    bm = 2048
    bs = b if isinstance(b, (list, tuple)) else [b]
    w = sum(x.shape[1] for x in bs)
    args = [p] + list(bs)
    in_specs = [pl.BlockSpec((bm, bm), lambda i, k: (i, k))]
    for x in bs:
        in_specs.append(pl.BlockSpec((bm, x.shape[1]), lambda i, k: (k, 0)))
    if bscale is not None:
        args.append(bscale)
        in_specs.append(pl.BlockSpec((bm, 1), lambda i, k: (k, 0)))
    for arr, mode in extras:
        args.append(arr)
        if mode == "i":
            in_specs.append(pl.BlockSpec((bm, arr.shape[1]),
                                         lambda i, k: (i, 0)))
        else:
            in_specs.append(pl.BlockSpec(arr.shape, lambda i, k: (0, 0)))
    out_shapes = [jax.ShapeDtypeStruct((N, ow), F32) for ow in out_widths]
    out_specs = [pl.BlockSpec((bm, ow), lambda i, k: (i, 0))
                 for ow in out_widths]
    res = pl.pallas_call(
        functools.partial(_adjmm_body, n_b=len(bs), n_extra=len(extras),
                          n_out=len(out_widths), epilogue=epilogue,
                          bscale=bscale is not None),
        grid=(N // bm, N // bm),
        in_specs=in_specs,
        out_specs=out_specs,
        out_shape=out_shapes,
        scratch_shapes=[pltpu.VMEM((bm, w), F32)],
    )(*args)
    return res


def kernel(x, adj_l, dec, edge_index, Wse1, bse1, Wse2, bse2, Wsd1, bsd1,
           Wsd2, bsd2, We1, be1, We2, be2, Wg3, Wg4, Wg5, weight1, weight2,
           cluster_layer):
    cflat = _build_count(edge_index)
    adj_bf = _cast_bf(adj_l)
    xn = _rownorm(x)
    wcat = jnp.concatenate([Wse1, We1], axis=1)
    h, u1 = _smallmm(x, wcat, 256)
    W1d = _simw(xn)

    C_bf, A1, dinv1 = _retile(cflat, W1d)

    Wse2p = jnp.pad(Wse2, ((0, 0), (0, 96)))
    bse1r = bse1.reshape(1, -1)

    def ep1(acc, ex):
        dinv, hh, bb, w2p = ex
        h1 = jnp.maximum(dinv * acc + dinv * dinv * hh + bb, 0.0)
        nrm = jnp.maximum(jnp.sqrt(jnp.sum(h1 * h1, axis=1, keepdims=True)),
                          1e-8)
        return h1 / nrm, jnp.dot(h1, w2p, preferred_element_type=F32)

    h1n, h2p = _adjmm(A1, h,
                      [(dinv1, "i"), (h, "i"), (bse1r, "full"),
                       (Wse2p, "full")],
                      ep1, [256, 128], bscale=dinv1)

    A2, dinv2 = _simadj(h1n, C_bf, a1=W1d)
    bse2p = jnp.pad(bse2, (0, 96)).reshape(1, -1)
    Wsd1p = jnp.pad(Wsd1, ((0, 96), (0, 0)))

    def ep2(acc, ex):
        dinv, hh, bb, wsd1p = ex
        e = dinv * acc + dinv * dinv * hh + bb
        return e[:, :32], jnp.dot(e, wsd1p, preferred_element_type=F32)

    emb32, g1 = _adjmm(A2, h2p,
                       [(dinv2, "i"), (h2p, "i"), (bse2p, "full"),
                        (Wsd1p, "full")],
                       ep2, [32, 256], bscale=dinv2)

    bias3 = jnp.concatenate([bsd1, be1]).reshape(1, -1)
    W6 = jnp.zeros((384, 384), F32)
    W6 = W6.at[:256, :256].set(Wsd2)
    W6 = W6.at[256:, 256:288].set(We2)

    def ep3(acc, ex):
        bb, w6 = ex
        t = jnp.maximum(acc + bb, 0.0)
        return (jnp.dot(t, w6, preferred_element_type=F32),)

    (B2,) = _adjmm(C_bf, [g1, u1], [(bias3, "full"), (W6, "full")],
                   ep3, [384])

    bias4 = jnp.concatenate([bsd2, be2, jnp.zeros((96,), F32)]).reshape(1, -1)
    W7z = jnp.zeros((384, 128), F32).at[256:288, 0:64].set(Wg3)
    W7d = jnp.zeros((128, 128), F32).at[:, 64:72].set(weight1)

    def ep4(acc, ex):
        bb, w7z, w7d, dcb = ex
        z2 = acc + bb
        p1 = (jnp.dot(z2, w7z, preferred_element_type=F32)
              + jnp.dot(dcb, w7d, preferred_element_type=F32))
        return z2[:, :256], z2[:, 256:288], p1

    z_hat_emb, h_enc, P1 = _adjmm(C_bf, B2,
                                  [(bias4, "full"), (W7z, "full"),
                                   (W7d, "full"), (dec, "i")],
                                  ep4, [256, 32, 128])

    W8 = jnp.zeros((128, 256), F32)
    W8 = W8.at[0:64, 0:128].set(Wg4)
    W8 = W8.at[64:72, 128:256].set(weight2)

    def ep5(acc, ex):
        (w8,) = ex
        colid = lax.broadcasted_iota(jnp.int32, acc.shape, 1)
        q1 = jnp.where(colid < 64, jnp.maximum(acc, 0.0), acc)
        return q1, jnp.dot(q1, w8, preferred_element_type=F32)

    Q1, P2 = _adjmm(adj_bf, P1, [(W8, "full")], ep5, [128, 256])

    W9 = jnp.zeros((256, 256), F32).at[0:128, :].set(Wg5)

    def ep6(acc, ex):
        w9, he, eg, q1ex = ex
        colid = lax.broadcasted_iota(jnp.int32, acc.shape, 1)
        q2 = jnp.where(colid < 128, jnp.maximum(acc, 0.0), acc)
        zc = jnp.concatenate([he, eg, q1ex[:, 64:72]], axis=1)
        return (jnp.dot(q2, w9, preferred_element_type=F32),
                acc[:, 128:], zc)

    p3, dec_hat, z = _adjmm(
        adj_bf, P2,
        [(W9, "full"), (h_enc, "i"), (emb32, "i"), (Q1, "i")],
        ep6, [256, 128, 72])

    cl_pad = jnp.pad(cluster_layer, ((0, 6), (0, 56)))

    def ep7(acc, ex):
        zc, cl = ex
        zp = jnp.concatenate([zc, jnp.zeros((zc.shape[0], 56), F32)], axis=1)
        z2s = jnp.sum(zp * zp, axis=1, keepdims=True)
        c2s = jnp.sum(cl * cl, axis=1)[None, :]
        cross = jnp.dot(zp, cl.T, preferred_element_type=F32)
        dist = z2s - 2.0 * cross + c2s
        qv = 1.0 / (1.0 + dist + 1e-8)
        qv = qv * qv / 2.0
        colid = lax.broadcasted_iota(jnp.int32, qv.shape, 1)
        qv = jnp.where(colid < 10, qv, 0.0)
        qn = qv / jnp.sum(qv, axis=1, keepdims=True)
        return acc, qn[:, :10]

    z_hat, q = _adjmm(adj_bf, p3, [(z, "i"), (cl_pad, "full")],
                      ep7, [256, 10])

    return (z_hat, q, z, z_hat_emb, dec_hat)


# speedup vs baseline: 31.3218x; 1.0189x over previous
import functools

import jax
import jax.numpy as jnp
from jax import lax
from jax.experimental import pallas as pl
from jax.experimental.pallas import tpu as pltpu
from jax.experimental.pallas import tpu_sc as plsc

N = 4096
E = 131072
THR = 0.1
BM = 1024
NB = N // BM
F32 = jnp.float32


NTILES = 16
EPT = E // NTILES
NCHUNKS = 16
CROWS = N // NCHUNKS
CWORDS = CROWS * N
DUMPW = 8192
ZN = CWORDS // NTILES
ZROWS = CROWS // NTILES
ZB = 8192
IDX_ROWS = EPT // 128
BF16 = jnp.bfloat16


def _count_body(edges_hbm, out_hbm, row_v, col_v, idx_a, idx_b, ones_v, zer_v,
                buf_sh, sem):
    cid = lax.axis_index("c")
    sid = lax.axis_index("s")

    ebase = pl.multiple_of(sid * EPT, 8)
    cp_r = pltpu.async_copy(edges_hbm.at[0, pl.ds(ebase, EPT)], row_v, sem)
    cp_c = pltpu.async_copy(edges_hbm.at[1, pl.ds(ebase, EPT)], col_v, sem)

    for j in range(8):
        ones_v[pl.ds(j * 16, 16)] = jnp.ones((16,), F32)

    def zbody(i, _):
        zer_v[pl.ds(i * 16, 16)] = jnp.zeros((16,), F32)
        return 0

    lax.fori_loop(0, ZB // 16, zbody, 0)
    cp_r.wait()
    cp_c.wait()

    lane = lax.iota(jnp.int32, 16)

    def compute_idx(idx_v, base):
        def ibody(i, _):
            for j in range(8):
                e0 = i * 128 + j * 16
                r = row_v[pl.ds(e0, 16)]
                c = col_v[pl.ds(e0, 16)]
                full = r * N + c - base
                valid = (full >= 0) & (full < CWORDS)
                dump = CWORDS + ((e0 + lane) & (DUMPW - 1))
                idx_v[i, pl.ds(j * 16, 16)] = jnp.where(valid, full, dump)
            return 0

        lax.fori_loop(0, IDX_ROWS, ibody, 0)

    def chunk_base(ch):
        return pl.multiple_of((cid * (NCHUNKS // 2) + ch) * CWORDS, 8)

    bufs = [idx_a, idx_b]
    compute_idx(bufs[0], chunk_base(0))
    out_cps = []

    for ch in range(NCHUNKS // 2):
        base = chunk_base(ch)
        for cp in out_cps:
            cp.wait()
        out_cps = []

        zcps = [
            pltpu.async_copy(
                zer_v,
                buf_sh.at[pl.ds(pl.multiple_of(sid * ZN + zz * ZB, 8), ZB)],
                sem)
            for zz in range(ZN // ZB)]

        @pl.when(sid == 0)
        def _zd():
            pltpu.sync_copy(zer_v, buf_sh.at[pl.ds(CWORDS, DUMPW)])

        for cp in zcps:
            cp.wait()

        plsc.subcore_barrier()

        idx_v = bufs[ch % 2]
        scps = [pltpu.async_copy(ones_v, buf_sh.at[idx_v.at[jj]], sem,
                                 add=True)
                for jj in range(IDX_ROWS)]
        if ch + 1 < NCHUNKS // 2:
            compute_idx(bufs[(ch + 1) % 2], chunk_base(ch + 1))
        for cp in scps:
            cp.wait()

        plsc.subcore_barrier()

        dst = pl.multiple_of(base + sid * ZN, 8)
        out_cps.append(pltpu.async_copy(
            buf_sh.at[pl.ds(pl.multiple_of(sid * ZN, 8), ZN)],
            out_hbm.at[pl.ds(dst, ZN)], sem))

    for cp in out_cps:
        cp.wait()


def _build_count(edge_index):
    mesh = plsc.VectorSubcoreMesh(core_axis_name="c", subcore_axis_name="s")
    fn = pl.kernel(
        _count_body,
        out_type=jax.ShapeDtypeStruct((N * N,), F32),
        mesh=mesh,
        scratch_types=[
            pltpu.VMEM((EPT,), jnp.int32),
            pltpu.VMEM((EPT,), jnp.int32),
            pltpu.VMEM((IDX_ROWS, 128), jnp.int32),
            pltpu.VMEM((IDX_ROWS, 128), jnp.int32),
            pltpu.VMEM((128,), F32),
            pltpu.VMEM((ZB,), F32),
            pltpu.VMEM_SHARED((CWORDS + DUMPW,), F32),
            pltpu.SemaphoreType.DMA,
        ],
    )
    return fn(edge_index)


def _retile_body(x_ref, w_ref, c_ref, a_ref, dinv_ref):
    c = x_ref[...].reshape(c_ref.shape)
    c_ref[...] = c.astype(BF16)
    a = w_ref[...].astype(F32) * c
    a_ref[...] = a.astype(BF16)
    dinv_ref[...] = lax.rsqrt(jnp.sum(a, axis=1, keepdims=True) + 1.0)


def _retile(cflat, w1d):
    rb = 512
    return pl.pallas_call(
        _retile_body,
        grid=(N // rb,),
        in_specs=[pl.BlockSpec((rb * N,), lambda i: (i,)),
                  pl.BlockSpec((rb, N), lambda i: (i, 0))],
        out_specs=[pl.BlockSpec((rb, N), lambda i: (i, 0)),
                   pl.BlockSpec((rb, N), lambda i: (i, 0)),
                   pl.BlockSpec((rb, 1), lambda i: (i, 0))],
        out_shape=[jax.ShapeDtypeStruct((N, N), BF16),
                   jax.ShapeDtypeStruct((N, N), BF16),
                   jax.ShapeDtypeStruct((N, 1), F32)],
    )(cflat, w1d)


def _simw_body(xi_ref, xj_ref, w_ref):
    i = pl.program_id(0)
    j = pl.program_id(1)
    s = jnp.dot(xi_ref[...], xj_ref[...].T, preferred_element_type=F32)
    ri = lax.broadcasted_iota(jnp.int32, s.shape, 0) + (i - j) * BM
    ci = lax.broadcasted_iota(jnp.int32, s.shape, 1)
    m = (s >= THR) & (ri != ci)
    w_ref[...] = jnp.where(m, s, 0.0).astype(BF16)


def _simw(xn):
    return pl.pallas_call(
        _simw_body,
        grid=(NB, NB),
        in_specs=[pl.BlockSpec((BM, xn.shape[1]), lambda i, j: (i, 0)),
                  pl.BlockSpec((BM, xn.shape[1]), lambda i, j: (j, 0))],
        out_specs=pl.BlockSpec((BM, BM), lambda i, j: (i, j)),
        out_shape=jax.ShapeDtypeStruct((N, N), BF16),
    )(xn, xn)


def _cast_body(x_ref, o_ref):
    o_ref[...] = x_ref[...].astype(BF16)


def _cast_bf(x):
    rb = 512
    return pl.pallas_call(
        _cast_body,
        grid=(N // rb,),
        in_specs=[pl.BlockSpec((rb, N), lambda i: (i, 0))],
        out_specs=pl.BlockSpec((rb, N), lambda i: (i, 0)),
        out_shape=jax.ShapeDtypeStruct((N, N), BF16),
    )(x)


def _mm_body(x_ref, w_ref, h_ref, u_ref, *, hsplit):
    hh = jnp.dot(x_ref[...], w_ref[...], preferred_element_type=F32)
    h_ref[...] = hh[:, :hsplit]
    u_ref[...] = hh[:, hsplit:].astype(BF16)


def _smallmm(x, w, hsplit):
    wtot = w.shape[1]
    return pl.pallas_call(
        functools.partial(_mm_body, hsplit=hsplit),
        grid=(N // BM,),
        in_specs=[pl.BlockSpec((BM, x.shape[1]), lambda i: (i, 0)),
                  pl.BlockSpec(w.shape, lambda i: (0, 0))],
        out_specs=[pl.BlockSpec((BM, hsplit), lambda i: (i, 0)),
                   pl.BlockSpec((BM, wtot - hsplit), lambda i: (i, 0))],
        out_shape=[jax.ShapeDtypeStruct((N, hsplit), F32),
                   jax.ShapeDtypeStruct((N, wtot - hsplit), BF16)],
    )(x, w)


def _rownorm_body(x_ref, o_ref):
    x = x_ref[...]
    nrm = jnp.maximum(jnp.sqrt(jnp.sum(x * x, axis=1, keepdims=True)), 1e-8)
    o_ref[...] = x / nrm


def _rownorm(x):
    n, d = x.shape
    return pl.pallas_call(
        _rownorm_body,
        grid=(n // BM,),
        in_specs=[pl.BlockSpec((BM, d), lambda i: (i, 0))],
        out_specs=pl.BlockSpec((BM, d), lambda i: (i, 0)),
        out_shape=jax.ShapeDtypeStruct((n, d), F32),
    )(x)


def _simadj_body(*refs, with_prev):
    idx = 0
    xi = refs[idx]; idx += 1
    xj = refs[idx]; idx += 1
    cref = refs[idx]; idx += 1
    if with_prev:
        a1_ref = refs[idx]; idx += 1
    a_ref = refs[idx]; idx += 1
    dinv_ref = refs[idx]; idx += 1
    acc_ref = refs[idx]

    i = pl.program_id(0)
    j = pl.program_id(1)
    s = jnp.dot(xi[...], xj[...].T, preferred_element_type=F32)
    ri = lax.broadcasted_iota(jnp.int32, (BM, BM), 0) + (i - j) * BM
    ci = lax.broadcasted_iota(jnp.int32, (BM, BM), 1)
    m = (s >= THR) & (ri != ci)
    if with_prev:
        m = m & (a1_ref[...] > 0)
    c = cref[...]
    if c.dtype != F32:
        c = c.astype(F32)
    a = jnp.where(m, s, 0.0) * c
    a_ref[...] = a.astype(jnp.bfloat16)

    @pl.when(j == 0)
    def _z():
        acc_ref[...] = jnp.zeros_like(acc_ref)

    acc_ref[...] += jnp.sum(a, axis=1, keepdims=True)

    @pl.when(j == NB - 1)
    def _f():
        dinv_ref[...] = lax.rsqrt(acc_ref[...] + 1.0)


def _simadj(xn, c, a1=None):
    with_prev = a1 is not None
    args = [xn, xn, c]
    in_specs = [pl.BlockSpec((BM, xn.shape[1]), lambda i, j: (i, 0)),
                pl.BlockSpec((BM, xn.shape[1]), lambda i, j: (j, 0)),
                pl.BlockSpec((BM, BM), lambda i, j: (i, j))]
    if with_prev:
        args.append(a1)
        in_specs.append(pl.BlockSpec((BM, BM), lambda i, j: (i, j)))
    out_shapes = [jax.ShapeDtypeStruct((N, N), jnp.bfloat16),
                  jax.ShapeDtypeStruct((N, 1), F32)]
    out_specs = [pl.BlockSpec((BM, BM), lambda i, j: (i, j)),
                 pl.BlockSpec((BM, 1), lambda i, j: (i, 0))]
    return pl.pallas_call(
        functools.partial(_simadj_body, with_prev=with_prev),
        grid=(NB, NB),
        in_specs=in_specs,
        out_specs=out_specs,
        out_shape=out_shapes,
        scratch_shapes=[pltpu.VMEM((BM, 1), F32)],
    )(*args)


def _adjmm_body(*refs, n_b, n_extra, n_out, epilogue, bscale):
    idx = 0
    p_ref = refs[idx]; idx += 1
    b_refs = refs[idx:idx + n_b]; idx += n_b
    if bscale:
        bs_ref = refs[idx]; idx += 1
    extra_refs = refs[idx:idx + n_extra]; idx += n_extra
    out_refs = refs[idx:idx + n_out]; idx += n_out
    acc_ref = refs[idx]

    k = pl.program_id(1)

    @pl.when(k == 0)
    def _z():
        acc_ref[...] = jnp.zeros_like(acc_ref)

    if n_b == 1:
        bb = b_refs[0][...]
    else:
        bb = jnp.concatenate([br[...] for br in b_refs], axis=1)
    if bscale:
        bb = bs_ref[...] * bb
    acc_ref[...] += jnp.dot(p_ref[...], bb.astype(jnp.bfloat16),
                            preferred_element_type=F32)

    @pl.when(k == pl.num_programs(1) - 1)
    def _f():
        outs = epilogue(acc_ref[...], [er[...] for er in extra_refs])
        for o_ref, o in zip(out_refs, outs):
            o_ref[...] = o


def _adjmm(p, b, extras, epilogue, out_widths, bscale=None, out_dtypes=None):
    bm = 2048
    bs = b if isinstance(b, (list, tuple)) else [b]
    w = sum(x.shape[1] for x in bs)
    args = [p] + list(bs)
    in_specs = [pl.BlockSpec((bm, bm), lambda i, k: (i, k))]
    for x in bs:
        in_specs.append(pl.BlockSpec((bm, x.shape[1]), lambda i, k: (k, 0)))
    if bscale is not None:
        args.append(bscale)
        in_specs.append(pl.BlockSpec((bm, 1), lambda i, k: (k, 0)))
    for arr, mode in extras:
        args.append(arr)
        if mode == "i":
            in_specs.append(pl.BlockSpec((bm, arr.shape[1]),
                                         lambda i, k: (i, 0)))
        else:
            in_specs.append(pl.BlockSpec(arr.shape, lambda i, k: (0, 0)))
    out_shapes = [jax.ShapeDtypeStruct((N, ow), dt)
                  for ow, dt in zip(out_widths, out_dtypes or
                                    [F32] * len(out_widths))]
    out_specs = [pl.BlockSpec((bm, ow), lambda i, k: (i, 0))
                 for ow in out_widths]
    res = pl.pallas_call(
        functools.partial(_adjmm_body, n_b=len(bs), n_extra=len(extras),
                          n_out=len(out_widths), epilogue=epilogue,
                          bscale=bscale is not None),
        grid=(N // bm, N // bm),
        in_specs=in_specs,
        out_specs=out_specs,
        out_shape=out_shapes,
        scratch_shapes=[pltpu.VMEM((bm, w), F32)],
    )(*args)
    return res


def kernel(x, adj_l, dec, edge_index, Wse1, bse1, Wse2, bse2, Wsd1, bsd1,
           Wsd2, bsd2, We1, be1, We2, be2, Wg3, Wg4, Wg5, weight1, weight2,
           cluster_layer):
    cflat = _build_count(edge_index)
    adj_bf = _cast_bf(adj_l)
    xn = _rownorm(x)
    wcat = jnp.concatenate([Wse1, We1], axis=1)
    h, u1 = _smallmm(x, wcat, 256)
    W1d = _simw(xn)

    C_bf, A1, dinv1 = _retile(cflat, W1d)

    Wse2p = jnp.pad(Wse2, ((0, 0), (0, 96)))
    bse1r = bse1.reshape(1, -1)

    def ep1(acc, ex):
        dinv, hh, bb, w2p = ex
        h1 = jnp.maximum(dinv * acc + dinv * dinv * hh + bb, 0.0)
        nrm = jnp.maximum(jnp.sqrt(jnp.sum(h1 * h1, axis=1, keepdims=True)),
                          1e-8)
        return h1 / nrm, jnp.dot(h1, w2p, preferred_element_type=F32)

    h1n, h2p = _adjmm(A1, h,
                      [(dinv1, "i"), (h, "i"), (bse1r, "full"),
                       (Wse2p, "full")],
                      ep1, [256, 128], bscale=dinv1)

    A2, dinv2 = _simadj(h1n, C_bf, a1=W1d)
    bse2p = jnp.pad(bse2, (0, 96)).reshape(1, -1)
    Wsd1p = jnp.pad(Wsd1, ((0, 96), (0, 0)))

    def ep2(acc, ex):
        dinv, hh, bb, wsd1p = ex
        e = dinv * acc + dinv * dinv * hh + bb
        return (e[:, :32],
                jnp.dot(e, wsd1p, preferred_element_type=F32).astype(BF16))

    emb32, g1 = _adjmm(A2, h2p,
                       [(dinv2, "i"), (h2p, "i"), (bse2p, "full"),
                        (Wsd1p, "full")],
                       ep2, [32, 256], bscale=dinv2,
                       out_dtypes=[F32, BF16])

    bias3 = jnp.concatenate([bsd1, be1]).reshape(1, -1)
    W6 = jnp.zeros((384, 384), F32)
    W6 = W6.at[:256, :256].set(Wsd2)
    W6 = W6.at[256:, 256:288].set(We2)

    def ep3(acc, ex):
        bb, w6 = ex
        t = jnp.maximum(acc + bb, 0.0)
        return (jnp.dot(t, w6, preferred_element_type=F32).astype(BF16),)

    (B2,) = _adjmm(C_bf, [g1, u1], [(bias3, "full"), (W6, "full")],
                   ep3, [384], out_dtypes=[BF16])

    bias4 = jnp.concatenate([bsd2, be2, jnp.zeros((96,), F32)]).reshape(1, -1)
    W7z = jnp.zeros((384, 128), F32).at[256:288, 0:64].set(Wg3)
    W7d = jnp.zeros((128, 128), F32).at[:, 64:72].set(weight1)

    def ep4(acc, ex):
        bb, w7z, w7d, dcb = ex
        z2 = acc + bb
        p1 = (jnp.dot(z2, w7z, preferred_element_type=F32)
              + jnp.dot(dcb, w7d, preferred_element_type=F32))
        return z2[:, :256], z2[:, 256:288], p1.astype(BF16)

    z_hat_emb, h_enc, P1 = _adjmm(C_bf, B2,
                                  [(bias4, "full"), (W7z, "full"),
                                   (W7d, "full"), (dec, "i")],
                                  ep4, [256, 32, 128],
                                  out_dtypes=[F32, F32, BF16])

    W8 = jnp.zeros((128, 256), F32)
    W8 = W8.at[0:64, 0:128].set(Wg4)
    W8 = W8.at[64:72, 128:256].set(weight2)

    def ep5(acc, ex):
        (w8,) = ex
        colid = lax.broadcasted_iota(jnp.int32, acc.shape, 1)
        q1 = jnp.where(colid < 64, jnp.maximum(acc, 0.0), acc)
        return q1, jnp.dot(q1, w8, preferred_element_type=F32).astype(BF16)

    Q1, P2 = _adjmm(adj_bf, P1, [(W8, "full")], ep5, [128, 256],
                    out_dtypes=[F32, BF16])

    W9 = jnp.zeros((256, 256), F32).at[0:128, :].set(Wg5)

    def ep6(acc, ex):
        w9, he, eg, q1ex = ex
        colid = lax.broadcasted_iota(jnp.int32, acc.shape, 1)
        q2 = jnp.where(colid < 128, jnp.maximum(acc, 0.0), acc)
        zc = jnp.concatenate([he, eg, q1ex[:, 64:72]], axis=1)
        return (jnp.dot(q2, w9, preferred_element_type=F32).astype(BF16),
                acc[:, 128:], zc)

    p3, dec_hat, z = _adjmm(
        adj_bf, P2,
        [(W9, "full"), (h_enc, "i"), (emb32, "i"), (Q1, "i")],
        ep6, [256, 128, 72], out_dtypes=[BF16, F32, F32])

    cl_pad = jnp.pad(cluster_layer, ((0, 6), (0, 56)))

    def ep7(acc, ex):
        zc, cl = ex
        zp = jnp.concatenate([zc, jnp.zeros((zc.shape[0], 56), F32)], axis=1)
        z2s = jnp.sum(zp * zp, axis=1, keepdims=True)
        c2s = jnp.sum(cl * cl, axis=1)[None, :]
        cross = jnp.dot(zp, cl.T, preferred_element_type=F32)
        dist = z2s - 2.0 * cross + c2s
        qv = 1.0 / (1.0 + dist + 1e-8)
        qv = qv * qv / 2.0
        colid = lax.broadcasted_iota(jnp.int32, qv.shape, 1)
        qv = jnp.where(colid < 10, qv, 0.0)
        qn = qv / jnp.sum(qv, axis=1, keepdims=True)
        return acc, qn[:, :10]

    z_hat, q = _adjmm(adj_bf, p3, [(z, "i"), (cl_pad, "full")],
                      ep7, [256, 10])

    return (z_hat, q, z, z_hat_emb, dec_hat)
